# Initial kernel scaffold; baseline (speedup 1.0000x reference)
import jax
import jax.numpy as jnp
from jax.experimental import pallas as pl


def kernel(x, edge_index, Wl0, bl0, Wr0, Wl1, bl1, Wr1, Wl2, bl2, Wr2, Wl3, bl3, Wr3):
    raise NotImplementedError("write your pallas kernel here")


# speedup vs baseline: 4.3942x; 4.3942x over previous
import functools

import jax
import jax.numpy as jnp
from jax import lax
from jax.experimental import pallas as pl
from jax.experimental.pallas import tpu as pltpu
from jax.experimental.pallas import tpu_sc as plsc

N = 10000
NP = 10240
D = 128
H = 64
E = 320000
CH = 128
NW = 32
K = 79
EP = NW * K * CH
RPT = NP // 16
ZR = 32


def _make_sc_agg(F: int):
    mesh = plsc.VectorSubcoreMesh(core_axis_name="c", subcore_axis_name="s")
    out_type = jax.ShapeDtypeStruct((2, NP, F), jnp.float32)
    scratch = [
        pltpu.VMEM((CH,), jnp.int32),
        pltpu.VMEM((CH,), jnp.int32),
        pltpu.VMEM((CH, F), jnp.float32),
        pltpu.VMEM((ZR, F), jnp.float32),
        pltpu.VMEM_SHARED((NP, F), jnp.float32),
        pltpu.SemaphoreType.DMA,
    ]

    def body(h_hbm, src_hbm, dst_hbm, out_hbm, src_v, dst_v, rows_v, zb_v,
             acc_sh, sem):
        c = lax.axis_index("c")
        s = lax.axis_index("s")
        w = c * 16 + s

        def zfill(i, carry):
            for j in range(F // 16):
                zb_v[i, pl.ds(j * 16, 16)] = jnp.zeros((16,), jnp.float32)
            return carry
        lax.fori_loop(0, ZR, zfill, 0)
        for t in range(RPT // ZR):
            pltpu.sync_copy(zb_v, acc_sh.at[pl.ds(s * RPT + t * ZR, ZR)])
        plsc.subcore_barrier()

        def chunk(k, carry):
            pltpu.sync_copy(src_hbm.at[w, k], src_v)
            pltpu.sync_copy(dst_hbm.at[w, k], dst_v)
            pltpu.async_copy(h_hbm.at[src_v], rows_v, sem).wait()
            pltpu.sync_copy(rows_v, acc_sh.at[dst_v], add=True)
            return carry
        lax.fori_loop(0, K, chunk, 0)
        plsc.subcore_barrier()

        pltpu.sync_copy(acc_sh.at[pl.ds(s * RPT, RPT)],
                        out_hbm.at[c, pl.ds(s * RPT, RPT)])

    return pl.kernel(body, mesh=mesh, out_type=out_type, scratch_types=scratch,
                     compiler_params=pltpu.CompilerParams(
                         use_tc_tiling_on_sc=False))


def _make_sc_counts():
    mesh = plsc.VectorSubcoreMesh(core_axis_name="c", subcore_axis_name="s")
    out_type = jax.ShapeDtypeStruct((2, NP, 16), jnp.float32)
    scratch = [
        pltpu.VMEM((CH,), jnp.int32),
        pltpu.VMEM((CH, 16), jnp.float32),
        pltpu.VMEM((ZR, 16), jnp.float32),
        pltpu.VMEM_SHARED((NP, 16), jnp.float32),
    ]

    def body(dst_hbm, cnt_hbm, dst_v, ones_v, zc_v, cnt_sh):
        c = lax.axis_index("c")
        s = lax.axis_index("s")
        w = c * 16 + s

        def ofill(i, carry):
            ones_v[i, :] = jnp.full((16,), 1.0, jnp.float32)
            return carry
        lax.fori_loop(0, CH, ofill, 0)

        def zcfill(i, carry):
            zc_v[i, :] = jnp.zeros((16,), jnp.float32)
            return carry
        lax.fori_loop(0, ZR, zcfill, 0)
        for t in range(RPT // ZR):
            pltpu.sync_copy(zc_v, cnt_sh.at[pl.ds(s * RPT + t * ZR, ZR)])
        plsc.subcore_barrier()

        def chunk(k, carry):
            pltpu.sync_copy(dst_hbm.at[w, k], dst_v)
            pltpu.sync_copy(ones_v, cnt_sh.at[dst_v], add=True)
            return carry
        lax.fori_loop(0, K, chunk, 0)
        plsc.subcore_barrier()

        pltpu.sync_copy(cnt_sh.at[pl.ds(s * RPT, RPT)],
                        cnt_hbm.at[c, pl.ds(s * RPT, RPT)])

    return pl.kernel(body, mesh=mesh, out_type=out_type, scratch_types=scratch,
                     compiler_params=pltpu.CompilerParams(
                         use_tc_tiling_on_sc=False))


_sc_counts = _make_sc_counts()
_sc64 = _make_sc_agg(H)
_sc128 = _make_sc_agg(D)


def _dotT(a, w):
    return lax.dot_general(a, w, (((1,), (1,)), ((), ())),
                           precision=lax.Precision.HIGHEST,
                           preferred_element_type=jnp.float32)


def _mean(p_ref, cp_ref):
    cnt = cp_ref[0, :, 0:1] + cp_ref[1, :, 0:1]
    inv = 1.0 / jnp.maximum(cnt, 1.0)
    return (p_ref[0] + p_ref[1]) * inv


R = 1024
G = NP // R


def _tc0_body(p_ref, cp_ref, x_ref, wl0_ref, bl0_ref, wr0_ref, wl1_ref,
              y0_ref, t1_ref):
    m = _mean(p_ref, cp_ref)
    y = _dotT(m, wl0_ref[...]) + bl0_ref[...][None, :] + _dotT(x_ref[...], wr0_ref[...])
    y = jnp.maximum(y, 0.0)
    y0_ref[...] = y
    t1_ref[...] = _dotT(y, wl1_ref[...])


_tc0 = pl.pallas_call(
    _tc0_body,
    grid=(G,),
    in_specs=[
        pl.BlockSpec((2, R, D), lambda i: (0, i, 0)),
        pl.BlockSpec((2, R, 16), lambda i: (0, i, 0)),
        pl.BlockSpec((R, D), lambda i: (i, 0)),
        pl.BlockSpec((D, D), lambda i: (0, 0)),
        pl.BlockSpec((D,), lambda i: (0,)),
        pl.BlockSpec((D, D), lambda i: (0, 0)),
        pl.BlockSpec((H, D), lambda i: (0, 0)),
    ],
    out_specs=[pl.BlockSpec((R, D), lambda i: (i, 0)),
               pl.BlockSpec((R, H), lambda i: (i, 0))],
    out_shape=[jax.ShapeDtypeStruct((NP, D), jnp.float32),
               jax.ShapeDtypeStruct((NP, H), jnp.float32)],
)


def _tc1_body(p_ref, cp_ref, y0_ref, bl1_ref, wr1_ref, h1_ref):
    m = _mean(p_ref, cp_ref)
    h1_ref[...] = m + bl1_ref[...][None, :] + _dotT(y0_ref[...], wr1_ref[...])


_tc1 = pl.pallas_call(
    _tc1_body,
    grid=(G,),
    in_specs=[
        pl.BlockSpec((2, R, H), lambda i: (0, i, 0)),
        pl.BlockSpec((2, R, 16), lambda i: (0, i, 0)),
        pl.BlockSpec((R, D), lambda i: (i, 0)),
        pl.BlockSpec((H,), lambda i: (0,)),
        pl.BlockSpec((H, D), lambda i: (0, 0)),
    ],
    out_specs=[pl.BlockSpec((R, H), lambda i: (i, 0))],
    out_shape=[jax.ShapeDtypeStruct((NP, H), jnp.float32)],
)


def _tc2_body(p_ref, cp_ref, h1_ref, wl2_ref, bl2_ref, wr2_ref, h2_ref):
    m = _mean(p_ref, cp_ref)
    y = _dotT(m, wl2_ref[...]) + bl2_ref[...][None, :] + _dotT(h1_ref[...], wr2_ref[...])
    h2_ref[...] = jnp.maximum(y, 0.0)


_tc2 = pl.pallas_call(
    _tc2_body,
    grid=(G,),
    in_specs=[
        pl.BlockSpec((2, R, H), lambda i: (0, i, 0)),
        pl.BlockSpec((2, R, 16), lambda i: (0, i, 0)),
        pl.BlockSpec((R, H), lambda i: (i, 0)),
        pl.BlockSpec((D, H), lambda i: (0, 0)),
        pl.BlockSpec((D,), lambda i: (0,)),
        pl.BlockSpec((D, H), lambda i: (0, 0)),
    ],
    out_specs=[pl.BlockSpec((R, D), lambda i: (i, 0))],
    out_shape=[jax.ShapeDtypeStruct((NP, D), jnp.float32)],
)


def _tc3_body(p_ref, cp_ref, h2_ref, wl3_ref, bl3_ref, wr3_ref, o_ref):
    m = _mean(p_ref, cp_ref)
    o_ref[...] = (_dotT(m, wl3_ref[...]) + bl3_ref[...][None, :]
                  + _dotT(h2_ref[...], wr3_ref[...]))


_tc3 = pl.pallas_call(
    _tc3_body,
    grid=(G,),
    in_specs=[
        pl.BlockSpec((2, R, D), lambda i: (0, i, 0)),
        pl.BlockSpec((2, R, 16), lambda i: (0, i, 0)),
        pl.BlockSpec((R, D), lambda i: (i, 0)),
        pl.BlockSpec((D, D), lambda i: (0, 0)),
        pl.BlockSpec((D,), lambda i: (0,)),
        pl.BlockSpec((D, D), lambda i: (0, 0)),
    ],
    out_specs=[pl.BlockSpec((R, D), lambda i: (i, 0))],
    out_shape=[jax.ShapeDtypeStruct((NP, D), jnp.float32)],
)


def _first(res):
    return res[0] if isinstance(res, (list, tuple)) else res


def kernel(x, edge_index, Wl0, bl0, Wr0, Wl1, bl1, Wr1, Wl2, bl2, Wr2,
           Wl3, bl3, Wr3):
    pad = EP - E
    src3 = jnp.concatenate(
        [edge_index[0], jnp.zeros((pad,), jnp.int32)]).reshape(NW, K, CH)
    dst3 = jnp.concatenate(
        [edge_index[1], jnp.full((pad,), N, jnp.int32)]).reshape(NW, K, CH)
    x_p = jnp.pad(x, ((0, NP - N), (0, 0)))

    cp = _first(_sc_counts(dst3))
    p0 = _first(_sc128(x_p, src3, dst3))
    y0, t1 = _tc0(p0, cp, x_p, Wl0, bl0, Wr0, Wl1)
    p1 = _first(_sc64(t1, src3, dst3))
    h1 = _first(_tc1(p1, cp, y0, bl1, Wr1))
    p2 = _first(_sc64(h1, src3, dst3))
    h2 = _first(_tc2(p2, cp, h1, Wl2, bl2, Wr2))
    p3 = _first(_sc128(h2, src3, dst3))
    out = _first(_tc3(p3, cp, h2, Wl3, bl3, Wr3))
    return out[:N]


# speedup vs baseline: 4.8329x; 1.0999x over previous
import functools

import jax
import jax.numpy as jnp
from jax import lax
from jax.experimental import pallas as pl
from jax.experimental.pallas import tpu as pltpu
from jax.experimental.pallas import tpu_sc as plsc

N = 10000
NP = 10240
D = 128
H = 64
E = 320000
CH = 128
NW = 32
K = 80
EP = NW * K * CH
RPT = NP // 16
ZR = 32


def _make_sc_agg(F: int):
    mesh = plsc.VectorSubcoreMesh(core_axis_name="c", subcore_axis_name="s")
    out_type = jax.ShapeDtypeStruct((2, NP, F), jnp.float32)
    scratch = [
        pltpu.VMEM((CH,), jnp.int32),
        pltpu.VMEM((CH,), jnp.int32),
        pltpu.VMEM((CH,), jnp.int32),
        pltpu.VMEM((CH,), jnp.int32),
        pltpu.VMEM((CH, F), jnp.float32),
        pltpu.VMEM((CH, F), jnp.float32),
        pltpu.VMEM((ZR, F), jnp.float32),
        pltpu.VMEM_SHARED((NP, F), jnp.float32),
        pltpu.SemaphoreType.DMA,
        pltpu.SemaphoreType.DMA,
    ]

    def body(h_hbm, src_hbm, dst_hbm, out_hbm, srcA, dstA, srcB, dstB,
             rowsA, rowsB, zb_v, acc_sh, semA, semB):
        c = lax.axis_index("c")
        s = lax.axis_index("s")
        w = c * 16 + s

        def zfill(i, carry):
            for j in range(F // 16):
                zb_v[i, pl.ds(j * 16, 16)] = jnp.zeros((16,), jnp.float32)
            return carry
        lax.fori_loop(0, ZR, zfill, 0)
        for t in range(RPT // ZR):
            pltpu.sync_copy(zb_v, acc_sh.at[pl.ds(s * RPT + t * ZR, ZR)])
        plsc.subcore_barrier()

        bufs = ((srcA, dstA, rowsA, semA), (srcB, dstB, rowsB, semB))

        def load_fire(k, src_v, dst_v, rows_v, sem):
            pltpu.sync_copy(src_hbm.at[w, k], src_v)
            pltpu.sync_copy(dst_hbm.at[w, k], dst_v)
            pltpu.async_copy(h_hbm.at[src_v], rows_v, sem)

        load_fire(0, *bufs[0])
        load_fire(1, *bufs[1])

        def pair(i, carry):
            for p in range(2):
                src_v, dst_v, rows_v, sem = bufs[p]
                k = 2 * i + p
                pltpu.make_async_copy(h_hbm.at[src_v], rows_v, sem).wait()
                pltpu.sync_copy(rows_v, acc_sh.at[dst_v], add=True)

                @pl.when(k + 2 < K)
                def _():
                    load_fire(k + 2, src_v, dst_v, rows_v, sem)
            return carry
        lax.fori_loop(0, K // 2, pair, 0)
        plsc.subcore_barrier()

        pltpu.sync_copy(acc_sh.at[pl.ds(s * RPT, RPT)],
                        out_hbm.at[c, pl.ds(s * RPT, RPT)])

    return pl.kernel(body, mesh=mesh, out_type=out_type, scratch_types=scratch,
                     compiler_params=pltpu.CompilerParams(
                         use_tc_tiling_on_sc=False))


def _make_sc_counts():
    mesh = plsc.VectorSubcoreMesh(core_axis_name="c", subcore_axis_name="s")
    out_type = jax.ShapeDtypeStruct((2, NP, 16), jnp.float32)
    scratch = [
        pltpu.VMEM((K, CH), jnp.int32),
        pltpu.VMEM((CH, 16), jnp.float32),
        pltpu.VMEM((ZR, 16), jnp.float32),
        pltpu.VMEM_SHARED((NP, 16), jnp.float32),
    ]

    def body(dst_hbm, cnt_hbm, dsts_v, ones_v, zc_v, cnt_sh):
        c = lax.axis_index("c")
        s = lax.axis_index("s")
        w = c * 16 + s

        def ofill(i, carry):
            ones_v[i, :] = jnp.full((16,), 1.0, jnp.float32)
            return carry
        lax.fori_loop(0, CH, ofill, 0)

        def zcfill(i, carry):
            zc_v[i, :] = jnp.zeros((16,), jnp.float32)
            return carry
        lax.fori_loop(0, ZR, zcfill, 0)
        for t in range(RPT // ZR):
            pltpu.sync_copy(zc_v, cnt_sh.at[pl.ds(s * RPT + t * ZR, ZR)])
        plsc.subcore_barrier()

        pltpu.sync_copy(dst_hbm.at[w], dsts_v)

        def chunk(k, carry):
            pltpu.sync_copy(ones_v, cnt_sh.at[dsts_v.at[k]], add=True)
            return carry
        lax.fori_loop(0, K, chunk, 0)
        plsc.subcore_barrier()

        pltpu.sync_copy(cnt_sh.at[pl.ds(s * RPT, RPT)],
                        cnt_hbm.at[c, pl.ds(s * RPT, RPT)])

    return pl.kernel(body, mesh=mesh, out_type=out_type, scratch_types=scratch,
                     compiler_params=pltpu.CompilerParams(
                         use_tc_tiling_on_sc=False))


_sc_counts = _make_sc_counts()
_sc64 = _make_sc_agg(H)
_sc128 = _make_sc_agg(D)


def _dotT(a, w):
    return lax.dot_general(a, w, (((1,), (1,)), ((), ())),
                           precision=lax.Precision.HIGHEST,
                           preferred_element_type=jnp.float32)


def _mean(p_ref, cp_ref):
    cnt = cp_ref[0, :, 0:1] + cp_ref[1, :, 0:1]
    inv = 1.0 / jnp.maximum(cnt, 1.0)
    return (p_ref[0] + p_ref[1]) * inv


R = 1024
G = NP // R


def _tc0_body(p_ref, cp_ref, x_ref, wl0_ref, bl0_ref, wr0_ref, wl1_ref,
              y0_ref, t1_ref):
    m = _mean(p_ref, cp_ref)
    y = _dotT(m, wl0_ref[...]) + bl0_ref[...][None, :] + _dotT(x_ref[...], wr0_ref[...])
    y = jnp.maximum(y, 0.0)
    y0_ref[...] = y
    t1_ref[...] = _dotT(y, wl1_ref[...])


_tc0 = pl.pallas_call(
    _tc0_body,
    grid=(G,),
    in_specs=[
        pl.BlockSpec((2, R, D), lambda i: (0, i, 0)),
        pl.BlockSpec((2, R, 16), lambda i: (0, i, 0)),
        pl.BlockSpec((R, D), lambda i: (i, 0)),
        pl.BlockSpec((D, D), lambda i: (0, 0)),
        pl.BlockSpec((D,), lambda i: (0,)),
        pl.BlockSpec((D, D), lambda i: (0, 0)),
        pl.BlockSpec((H, D), lambda i: (0, 0)),
    ],
    out_specs=[pl.BlockSpec((R, D), lambda i: (i, 0)),
               pl.BlockSpec((R, H), lambda i: (i, 0))],
    out_shape=[jax.ShapeDtypeStruct((NP, D), jnp.float32),
               jax.ShapeDtypeStruct((NP, H), jnp.float32)],
)


def _tc1_body(p_ref, cp_ref, y0_ref, bl1_ref, wr1_ref, h1_ref):
    m = _mean(p_ref, cp_ref)
    h1_ref[...] = m + bl1_ref[...][None, :] + _dotT(y0_ref[...], wr1_ref[...])


_tc1 = pl.pallas_call(
    _tc1_body,
    grid=(G,),
    in_specs=[
        pl.BlockSpec((2, R, H), lambda i: (0, i, 0)),
        pl.BlockSpec((2, R, 16), lambda i: (0, i, 0)),
        pl.BlockSpec((R, D), lambda i: (i, 0)),
        pl.BlockSpec((H,), lambda i: (0,)),
        pl.BlockSpec((H, D), lambda i: (0, 0)),
    ],
    out_specs=[pl.BlockSpec((R, H), lambda i: (i, 0))],
    out_shape=[jax.ShapeDtypeStruct((NP, H), jnp.float32)],
)


def _tc2_body(p_ref, cp_ref, h1_ref, wl2_ref, bl2_ref, wr2_ref, h2_ref):
    m = _mean(p_ref, cp_ref)
    y = _dotT(m, wl2_ref[...]) + bl2_ref[...][None, :] + _dotT(h1_ref[...], wr2_ref[...])
    h2_ref[...] = jnp.maximum(y, 0.0)


_tc2 = pl.pallas_call(
    _tc2_body,
    grid=(G,),
    in_specs=[
        pl.BlockSpec((2, R, H), lambda i: (0, i, 0)),
        pl.BlockSpec((2, R, 16), lambda i: (0, i, 0)),
        pl.BlockSpec((R, H), lambda i: (i, 0)),
        pl.BlockSpec((D, H), lambda i: (0, 0)),
        pl.BlockSpec((D,), lambda i: (0,)),
        pl.BlockSpec((D, H), lambda i: (0, 0)),
    ],
    out_specs=[pl.BlockSpec((R, D), lambda i: (i, 0))],
    out_shape=[jax.ShapeDtypeStruct((NP, D), jnp.float32)],
)


def _tc3_body(p_ref, cp_ref, h2_ref, wl3_ref, bl3_ref, wr3_ref, o_ref):
    m = _mean(p_ref, cp_ref)
    o_ref[...] = (_dotT(m, wl3_ref[...]) + bl3_ref[...][None, :]
                  + _dotT(h2_ref[...], wr3_ref[...]))


_tc3 = pl.pallas_call(
    _tc3_body,
    grid=(G,),
    in_specs=[
        pl.BlockSpec((2, R, D), lambda i: (0, i, 0)),
        pl.BlockSpec((2, R, 16), lambda i: (0, i, 0)),
        pl.BlockSpec((R, D), lambda i: (i, 0)),
        pl.BlockSpec((D, D), lambda i: (0, 0)),
        pl.BlockSpec((D,), lambda i: (0,)),
        pl.BlockSpec((D, D), lambda i: (0, 0)),
    ],
    out_specs=[pl.BlockSpec((R, D), lambda i: (i, 0))],
    out_shape=[jax.ShapeDtypeStruct((NP, D), jnp.float32)],
)


def _first(res):
    return res[0] if isinstance(res, (list, tuple)) else res


def kernel(x, edge_index, Wl0, bl0, Wr0, Wl1, bl1, Wr1, Wl2, bl2, Wr2,
           Wl3, bl3, Wr3):
    pad = EP - E
    src3 = jnp.concatenate(
        [edge_index[0], jnp.zeros((pad,), jnp.int32)]).reshape(NW, K, CH)
    dst3 = jnp.concatenate(
        [edge_index[1], jnp.full((pad,), N, jnp.int32)]).reshape(NW, K, CH)
    x_p = jnp.pad(x, ((0, NP - N), (0, 0)))

    cp = _first(_sc_counts(dst3))
    p0 = _first(_sc128(x_p, src3, dst3))
    y0, t1 = _tc0(p0, cp, x_p, Wl0, bl0, Wr0, Wl1)
    p1 = _first(_sc64(t1, src3, dst3))
    h1 = _first(_tc1(p1, cp, y0, bl1, Wr1))
    p2 = _first(_sc64(h1, src3, dst3))
    h2 = _first(_tc2(p2, cp, h1, Wl2, bl2, Wr2))
    p3 = _first(_sc128(h2, src3, dst3))
    out = _first(_tc3(p3, cp, h2, Wl3, bl3, Wr3))
    return out[:N]


# speedup vs baseline: 8.3970x; 1.7375x over previous
import functools

import jax
import jax.numpy as jnp
from jax import lax
from jax.experimental import pallas as pl
from jax.experimental.pallas import tpu as pltpu
from jax.experimental.pallas import tpu_sc as plsc

N = 10000
NP = 10240
D = 128
H = 64
E = 320000
CH = 128
NW = 32
K = 80
EP = NW * K * CH
GC = 4
NG = K // GC
RPT = NP // 16
ZR = 32


def _make_sc_agg64():
    F = H
    mesh = plsc.VectorSubcoreMesh(core_axis_name="c", subcore_axis_name="s")
    out_type = jax.ShapeDtypeStruct((2, NP, F), jnp.float32)
    scratch = [
        pltpu.VMEM((GC, CH), jnp.int32),
        pltpu.VMEM((GC, CH), jnp.int32),
        pltpu.VMEM((GC, CH), jnp.int32),
        pltpu.VMEM((GC, CH), jnp.int32),
        pltpu.VMEM((CH, F), jnp.float32),
        pltpu.VMEM((CH, F), jnp.float32),
        pltpu.VMEM((ZR, F), jnp.float32),
        pltpu.VMEM_SHARED((NP, F), jnp.float32),
        pltpu.VMEM_SHARED((NP, F), jnp.float32),
        pltpu.SemaphoreType.DMA,
        pltpu.SemaphoreType.DMA,
        pltpu.SemaphoreType.DMA,
    ]

    def body(h_hbm, src_hbm, dst_hbm, out_hbm, srcg0, dstg0, srcg1, dstg1,
             rows0, rows1, zb_v, feat_sh, acc_sh, gsem0, gsem1, zsem):
        c = lax.axis_index("c")
        s = lax.axis_index("s")
        w = c * 16 + s

        pltpu.async_copy(h_hbm.at[pl.ds(s * RPT, RPT)],
                         feat_sh.at[pl.ds(s * RPT, RPT)], zsem)
        def zfill(i, carry):
            for j in range(F // 16):
                zb_v[i, pl.ds(j * 16, 16)] = jnp.zeros((16,), jnp.float32)
            return carry
        lax.fori_loop(0, ZR, zfill, 0)
        for t in range(RPT // ZR):
            pltpu.async_copy(zb_v, acc_sh.at[pl.ds(s * RPT + t * ZR, ZR)], zsem)
        pltpu.make_async_copy(h_hbm.at[pl.ds(s * RPT, RPT)],
                              feat_sh.at[pl.ds(s * RPT, RPT)], zsem).wait()
        for t in range(RPT // ZR):
            pltpu.make_async_copy(
                zb_v, acc_sh.at[pl.ds(s * RPT, ZR)], zsem).wait()
        plsc.subcore_barrier()

        idxbufs = ((srcg0, dstg0), (srcg1, dstg1))
        rowsb = (rows0, rows1)
        gsems = (gsem0, gsem1)

        pltpu.sync_copy(src_hbm.at[w, pl.ds(0, GC)], srcg0)
        pltpu.sync_copy(dst_hbm.at[w, pl.ds(0, GC)], dstg0)
        pltpu.async_copy(feat_sh.at[srcg0.at[0]], rows0, gsem0)
        pltpu.async_copy(feat_sh.at[srcg0.at[1]], rows1, gsem1)

        def iter_body(i, carry):
            for gp in range(2):
                srcg, dstg = idxbufs[gp]
                nsrcg, ndstg = idxbufs[1 - gp]
                g = 2 * i + gp
                for j in range(GC):
                    p = j % 2
                    k = g * GC + j
                    rows_v = rowsb[p]
                    pltpu.make_async_copy(
                        feat_sh.at[srcg.at[j]], rows_v, gsems[p]).wait()
                    pltpu.sync_copy(rows_v, acc_sh.at[dstg.at[j]], add=True)

                    if j == 2:
                        @pl.when(g + 1 < NG)
                        def _():
                            pltpu.sync_copy(
                                src_hbm.at[w, pl.ds((g + 1) * GC, GC)], nsrcg)
                            pltpu.sync_copy(
                                dst_hbm.at[w, pl.ds((g + 1) * GC, GC)], ndstg)

                    @pl.when(k + 2 < K)
                    def _():
                        if j < GC - 2:
                            pltpu.async_copy(
                                feat_sh.at[srcg.at[j + 2]], rows_v, gsems[p])
                        else:
                            pltpu.async_copy(
                                feat_sh.at[nsrcg.at[j + 2 - GC]], rows_v,
                                gsems[p])
            return carry
        lax.fori_loop(0, NG // 2, iter_body, 0)
        plsc.subcore_barrier()

        pltpu.sync_copy(acc_sh.at[pl.ds(s * RPT, RPT)],
                        out_hbm.at[c, pl.ds(s * RPT, RPT)])

    return pl.kernel(body, mesh=mesh, out_type=out_type, scratch_types=scratch,
                     compiler_params=pltpu.CompilerParams(
                         use_tc_tiling_on_sc=False))


def _make_sc_counts():
    mesh = plsc.VectorSubcoreMesh(core_axis_name="c", subcore_axis_name="s")
    out_type = jax.ShapeDtypeStruct((2, NP, 16), jnp.float32)
    scratch = [
        pltpu.VMEM((K, CH), jnp.int32),
        pltpu.VMEM((CH, 16), jnp.float32),
        pltpu.VMEM((ZR, 16), jnp.float32),
        pltpu.VMEM_SHARED((NP, 16), jnp.float32),
    ]

    def body(dst_hbm, cnt_hbm, dsts_v, ones_v, zc_v, cnt_sh):
        c = lax.axis_index("c")
        s = lax.axis_index("s")
        w = c * 16 + s

        def ofill(i, carry):
            ones_v[i, :] = jnp.full((16,), 1.0, jnp.float32)
            return carry
        lax.fori_loop(0, CH, ofill, 0)

        def zcfill(i, carry):
            zc_v[i, :] = jnp.zeros((16,), jnp.float32)
            return carry
        lax.fori_loop(0, ZR, zcfill, 0)
        for t in range(RPT // ZR):
            pltpu.sync_copy(zc_v, cnt_sh.at[pl.ds(s * RPT + t * ZR, ZR)])
        plsc.subcore_barrier()

        pltpu.sync_copy(dst_hbm.at[w], dsts_v)

        def chunk(k, carry):
            pltpu.sync_copy(ones_v, cnt_sh.at[dsts_v.at[k]], add=True)
            return carry
        lax.fori_loop(0, K, chunk, 0)
        plsc.subcore_barrier()

        pltpu.sync_copy(cnt_sh.at[pl.ds(s * RPT, RPT)],
                        cnt_hbm.at[c, pl.ds(s * RPT, RPT)])

    return pl.kernel(body, mesh=mesh, out_type=out_type, scratch_types=scratch,
                     compiler_params=pltpu.CompilerParams(
                         use_tc_tiling_on_sc=False))


_sc_counts = _make_sc_counts()
_sc64 = _make_sc_agg64()


def _dotT(a, w):
    return lax.dot_general(a, w, (((1,), (1,)), ((), ())),
                           precision=lax.Precision.HIGHEST,
                           preferred_element_type=jnp.float32)


def _inv_cnt(cp_ref):
    cnt = cp_ref[0, :, 0:1] + cp_ref[1, :, 0:1]
    return 1.0 / jnp.maximum(cnt, 1.0)


def _mean1(p_ref, inv):
    return (p_ref[0] + p_ref[1]) * inv


R = 1024
G = NP // R

_vspec = pl.BlockSpec((2, R, H), lambda i: (0, i, 0))
_cspec = pl.BlockSpec((2, R, 16), lambda i: (0, i, 0))


def _tc0_body(pa_ref, pb_ref, cp_ref, x_ref, wl0_ref, bl0_ref, wr0_ref,
              wl1_ref, y0_ref, t1_ref):
    inv = _inv_cnt(cp_ref)
    ma = _mean1(pa_ref, inv)
    mb = _mean1(pb_ref, inv)
    y = (_dotT(ma, wl0_ref[:, :H]) + _dotT(mb, wl0_ref[:, H:])
         + bl0_ref[...][None, :] + _dotT(x_ref[...], wr0_ref[...]))
    y = jnp.maximum(y, 0.0)
    y0_ref[...] = y
    t1_ref[...] = _dotT(y, wl1_ref[...])


_tc0 = pl.pallas_call(
    _tc0_body,
    grid=(G,),
    in_specs=[
        _vspec,
        _vspec,
        _cspec,
        pl.BlockSpec((R, D), lambda i: (i, 0)),
        pl.BlockSpec((D, D), lambda i: (0, 0)),
        pl.BlockSpec((D,), lambda i: (0,)),
        pl.BlockSpec((D, D), lambda i: (0, 0)),
        pl.BlockSpec((H, D), lambda i: (0, 0)),
    ],
    out_specs=[pl.BlockSpec((R, D), lambda i: (i, 0)),
               pl.BlockSpec((R, H), lambda i: (i, 0))],
    out_shape=[jax.ShapeDtypeStruct((NP, D), jnp.float32),
               jax.ShapeDtypeStruct((NP, H), jnp.float32)],
)


def _tc1_body(p_ref, cp_ref, y0_ref, bl1_ref, wr1_ref, h1_ref):
    m = _mean1(p_ref, _inv_cnt(cp_ref))
    h1_ref[...] = m + bl1_ref[...][None, :] + _dotT(y0_ref[...], wr1_ref[...])


_tc1 = pl.pallas_call(
    _tc1_body,
    grid=(G,),
    in_specs=[
        _vspec,
        _cspec,
        pl.BlockSpec((R, D), lambda i: (i, 0)),
        pl.BlockSpec((H,), lambda i: (0,)),
        pl.BlockSpec((H, D), lambda i: (0, 0)),
    ],
    out_specs=[pl.BlockSpec((R, H), lambda i: (i, 0))],
    out_shape=[jax.ShapeDtypeStruct((NP, H), jnp.float32)],
)


def _tc2_body(p_ref, cp_ref, h1_ref, wl2_ref, bl2_ref, wr2_ref,
              h2a_ref, h2b_ref):
    m = _mean1(p_ref, _inv_cnt(cp_ref))
    y = (_dotT(m, wl2_ref[...]) + bl2_ref[...][None, :]
         + _dotT(h1_ref[...], wr2_ref[...]))
    y = jnp.maximum(y, 0.0)
    h2a_ref[...] = y[:, :H]
    h2b_ref[...] = y[:, H:]


_tc2 = pl.pallas_call(
    _tc2_body,
    grid=(G,),
    in_specs=[
        _vspec,
        _cspec,
        pl.BlockSpec((R, H), lambda i: (i, 0)),
        pl.BlockSpec((D, H), lambda i: (0, 0)),
        pl.BlockSpec((D,), lambda i: (0,)),
        pl.BlockSpec((D, H), lambda i: (0, 0)),
    ],
    out_specs=[pl.BlockSpec((R, H), lambda i: (i, 0)),
               pl.BlockSpec((R, H), lambda i: (i, 0))],
    out_shape=[jax.ShapeDtypeStruct((NP, H), jnp.float32),
               jax.ShapeDtypeStruct((NP, H), jnp.float32)],
)


def _tc3_body(pa_ref, pb_ref, cp_ref, h2a_ref, h2b_ref, wl3_ref, bl3_ref,
              wr3_ref, o_ref):
    inv = _inv_cnt(cp_ref)
    ma = _mean1(pa_ref, inv)
    mb = _mean1(pb_ref, inv)
    o_ref[...] = (_dotT(ma, wl3_ref[:, :H]) + _dotT(mb, wl3_ref[:, H:])
                  + bl3_ref[...][None, :]
                  + _dotT(h2a_ref[...], wr3_ref[:, :H])
                  + _dotT(h2b_ref[...], wr3_ref[:, H:]))


_tc3 = pl.pallas_call(
    _tc3_body,
    grid=(G,),
    in_specs=[
        _vspec,
        _vspec,
        _cspec,
        pl.BlockSpec((R, H), lambda i: (i, 0)),
        pl.BlockSpec((R, H), lambda i: (i, 0)),
        pl.BlockSpec((D, D), lambda i: (0, 0)),
        pl.BlockSpec((D,), lambda i: (0,)),
        pl.BlockSpec((D, D), lambda i: (0, 0)),
    ],
    out_specs=[pl.BlockSpec((R, D), lambda i: (i, 0))],
    out_shape=[jax.ShapeDtypeStruct((NP, D), jnp.float32)],
)


def _first(res):
    return res[0] if isinstance(res, (list, tuple)) else res


def kernel(x, edge_index, Wl0, bl0, Wr0, Wl1, bl1, Wr1, Wl2, bl2, Wr2,
           Wl3, bl3, Wr3):
    pad = EP - E
    src3 = jnp.concatenate(
        [edge_index[0], jnp.zeros((pad,), jnp.int32)]).reshape(NW, K, CH)
    dst3 = jnp.concatenate(
        [edge_index[1], jnp.full((pad,), N, jnp.int32)]).reshape(NW, K, CH)
    x_p = jnp.pad(x, ((0, NP - N), (0, 0)))
    xa = x_p[:, :H]
    xb = x_p[:, H:]

    cp = _first(_sc_counts(dst3))
    p0a = _first(_sc64(xa, src3, dst3))
    p0b = _first(_sc64(xb, src3, dst3))
    y0, t1 = _tc0(p0a, p0b, cp, x_p, Wl0, bl0, Wr0, Wl1)
    p1 = _first(_sc64(t1, src3, dst3))
    h1 = _first(_tc1(p1, cp, y0, bl1, Wr1))
    p2 = _first(_sc64(h1, src3, dst3))
    h2a, h2b = _tc2(p2, cp, h1, Wl2, bl2, Wr2)
    p3a = _first(_sc64(h2a, src3, dst3))
    p3b = _first(_sc64(h2b, src3, dst3))
    out = _first(_tc3(p3a, p3b, cp, h2a, h2b, Wl3, bl3, Wr3))
    return out[:N]


# speedup vs baseline: 10.4014x; 1.2387x over previous
import functools

import jax
import jax.numpy as jnp
from jax import lax
from jax.experimental import pallas as pl
from jax.experimental.pallas import tpu as pltpu
from jax.experimental.pallas import tpu_sc as plsc

N = 10000
NP = 10240
D = 128
H = 64
E = 320000
CH = 128
NW = 32
K = 80
EP = NW * K * CH
GC = 4
NG = K // GC
RPT = NP // 16
ZR = 32


def _make_sc_agg64():
    F = H
    mesh = plsc.VectorSubcoreMesh(core_axis_name="c", subcore_axis_name="s")
    out_type = jax.ShapeDtypeStruct((2, NP, F), jnp.float32)
    scratch = [
        pltpu.VMEM((GC, CH), jnp.int32),
        pltpu.VMEM((GC, CH), jnp.int32),
        pltpu.VMEM((GC, CH), jnp.int32),
        pltpu.VMEM((GC, CH), jnp.int32),
        pltpu.VMEM((CH, F), jnp.float32),
        pltpu.VMEM((CH, F), jnp.float32),
        pltpu.VMEM((CH, F), jnp.float32),
        pltpu.VMEM((CH, F), jnp.float32),
        pltpu.VMEM((ZR, F), jnp.float32),
        pltpu.VMEM_SHARED((NP, F), jnp.float32),
        pltpu.VMEM_SHARED((NP, F), jnp.float32),
        pltpu.SemaphoreType.DMA,
        pltpu.SemaphoreType.DMA,
        pltpu.SemaphoreType.DMA,
        pltpu.SemaphoreType.DMA,
        pltpu.SemaphoreType.DMA,
        pltpu.SemaphoreType.DMA,
        pltpu.SemaphoreType.DMA,
        pltpu.SemaphoreType.DMA,
        pltpu.SemaphoreType.DMA,
    ]

    def body(h_hbm, src_hbm, dst_hbm, out_hbm, srcg0, dstg0, srcg1, dstg1,
             rows0, rows1, rows2, rows3, zb_v, feat_sh, acc_sh,
             gsem0, gsem1, gsem2, gsem3, ssem0, ssem1, ssem2, ssem3, zsem):
        c = lax.axis_index("c")
        s = lax.axis_index("s")
        w = c * 16 + s

        pltpu.async_copy(h_hbm.at[pl.ds(s * RPT, RPT)],
                         feat_sh.at[pl.ds(s * RPT, RPT)], zsem)
        def zfill(i, carry):
            for j in range(F // 16):
                zb_v[i, pl.ds(j * 16, 16)] = jnp.zeros((16,), jnp.float32)
            return carry
        lax.fori_loop(0, ZR, zfill, 0)
        for t in range(RPT // ZR):
            pltpu.async_copy(zb_v, acc_sh.at[pl.ds(s * RPT + t * ZR, ZR)], zsem)
        pltpu.make_async_copy(h_hbm.at[pl.ds(s * RPT, RPT)],
                              feat_sh.at[pl.ds(s * RPT, RPT)], zsem).wait()
        for t in range(RPT // ZR):
            pltpu.make_async_copy(
                zb_v, acc_sh.at[pl.ds(s * RPT, ZR)], zsem).wait()
        plsc.subcore_barrier()

        idxbufs = ((srcg0, dstg0), (srcg1, dstg1))
        rowsb = (rows0, rows1, rows2, rows3)
        gsems = (gsem0, gsem1, gsem2, gsem3)
        ssems = (ssem0, ssem1, ssem2, ssem3)

        pltpu.sync_copy(src_hbm.at[w, pl.ds(0, GC)], srcg0)
        pltpu.sync_copy(dst_hbm.at[w, pl.ds(0, GC)], dstg0)
        pltpu.async_copy(feat_sh.at[srcg0.at[0]], rows0, gsem0)
        pltpu.async_copy(feat_sh.at[srcg0.at[1]], rows1, gsem1)

        def iter_body(i, carry):
            for gp in range(2):
                srcg, dstg = idxbufs[gp]
                nsrcg, ndstg = idxbufs[1 - gp]
                g = 2 * i + gp
                for j in range(GC):
                    k = g * GC + j
                    rows_v = rowsb[j]
                    nx = (j + 2) % 4
                    pltpu.make_async_copy(
                        feat_sh.at[srcg.at[j]], rows_v, gsems[j]).wait()
                    pltpu.async_copy(
                        rows_v, acc_sh.at[dstg.at[j]], ssems[j], add=True)

                    @pl.when(k >= 2)
                    def _():
                        if j < 2:
                            pltpu.make_async_copy(
                                rowsb[nx], acc_sh.at[ndstg.at[nx]],
                                ssems[nx]).wait()
                        else:
                            pltpu.make_async_copy(
                                rowsb[nx], acc_sh.at[dstg.at[nx]],
                                ssems[nx]).wait()

                    if j == 2:
                        @pl.when(g + 1 < NG)
                        def _():
                            pltpu.sync_copy(
                                src_hbm.at[w, pl.ds((g + 1) * GC, GC)], nsrcg)
                            pltpu.sync_copy(
                                dst_hbm.at[w, pl.ds((g + 1) * GC, GC)], ndstg)

                    @pl.when(k + 2 < K)
                    def _():
                        if j < GC - 2:
                            pltpu.async_copy(
                                feat_sh.at[srcg.at[j + 2]], rowsb[nx],
                                gsems[nx])
                        else:
                            pltpu.async_copy(
                                feat_sh.at[nsrcg.at[j + 2 - GC]], rowsb[nx],
                                gsems[nx])
            return carry
        lax.fori_loop(0, NG // 2, iter_body, 0)
        pltpu.make_async_copy(rows2, acc_sh.at[dstg1.at[2]], ssem2).wait()
        pltpu.make_async_copy(rows3, acc_sh.at[dstg1.at[3]], ssem3).wait()
        plsc.subcore_barrier()

        pltpu.sync_copy(acc_sh.at[pl.ds(s * RPT, RPT)],
                        out_hbm.at[c, pl.ds(s * RPT, RPT)])

    return pl.kernel(body, mesh=mesh, out_type=out_type, scratch_types=scratch,
                     compiler_params=pltpu.CompilerParams(
                         use_tc_tiling_on_sc=False))


def _make_sc_counts():
    mesh = plsc.VectorSubcoreMesh(core_axis_name="c", subcore_axis_name="s")
    out_type = jax.ShapeDtypeStruct((2, NP, 16), jnp.float32)
    scratch = [
        pltpu.VMEM((K, CH), jnp.int32),
        pltpu.VMEM((CH, 16), jnp.float32),
        pltpu.VMEM((ZR, 16), jnp.float32),
        pltpu.VMEM_SHARED((NP, 16), jnp.float32),
    ]

    def body(dst_hbm, cnt_hbm, dsts_v, ones_v, zc_v, cnt_sh):
        c = lax.axis_index("c")
        s = lax.axis_index("s")
        w = c * 16 + s

        def ofill(i, carry):
            ones_v[i, :] = jnp.full((16,), 1.0, jnp.float32)
            return carry
        lax.fori_loop(0, CH, ofill, 0)

        def zcfill(i, carry):
            zc_v[i, :] = jnp.zeros((16,), jnp.float32)
            return carry
        lax.fori_loop(0, ZR, zcfill, 0)
        for t in range(RPT // ZR):
            pltpu.sync_copy(zc_v, cnt_sh.at[pl.ds(s * RPT + t * ZR, ZR)])
        plsc.subcore_barrier()

        pltpu.sync_copy(dst_hbm.at[w], dsts_v)

        def chunk(k, carry):
            pltpu.sync_copy(ones_v, cnt_sh.at[dsts_v.at[k]], add=True)
            return carry
        lax.fori_loop(0, K, chunk, 0)
        plsc.subcore_barrier()

        pltpu.sync_copy(cnt_sh.at[pl.ds(s * RPT, RPT)],
                        cnt_hbm.at[c, pl.ds(s * RPT, RPT)])

    return pl.kernel(body, mesh=mesh, out_type=out_type, scratch_types=scratch,
                     compiler_params=pltpu.CompilerParams(
                         use_tc_tiling_on_sc=False))


_sc_counts = _make_sc_counts()
_sc64 = _make_sc_agg64()


def _dotT(a, w):
    return lax.dot_general(a, w, (((1,), (1,)), ((), ())),
                           precision=lax.Precision.HIGHEST,
                           preferred_element_type=jnp.float32)


def _inv_cnt(cp_ref):
    cnt = cp_ref[0, :, 0:1] + cp_ref[1, :, 0:1]
    return 1.0 / jnp.maximum(cnt, 1.0)


def _mean1(p_ref, inv):
    return (p_ref[0] + p_ref[1]) * inv


R = 1024
G = NP // R

_vspec = pl.BlockSpec((2, R, H), lambda i: (0, i, 0))
_cspec = pl.BlockSpec((2, R, 16), lambda i: (0, i, 0))


def _tc0_body(pa_ref, pb_ref, cp_ref, x_ref, wl0_ref, bl0_ref, wr0_ref,
              wl1_ref, y0_ref, t1_ref):
    inv = _inv_cnt(cp_ref)
    ma = _mean1(pa_ref, inv)
    mb = _mean1(pb_ref, inv)
    y = (_dotT(ma, wl0_ref[:, :H]) + _dotT(mb, wl0_ref[:, H:])
         + bl0_ref[...][None, :] + _dotT(x_ref[...], wr0_ref[...]))
    y = jnp.maximum(y, 0.0)
    y0_ref[...] = y
    t1_ref[...] = _dotT(y, wl1_ref[...])


_tc0 = pl.pallas_call(
    _tc0_body,
    grid=(G,),
    in_specs=[
        _vspec,
        _vspec,
        _cspec,
        pl.BlockSpec((R, D), lambda i: (i, 0)),
        pl.BlockSpec((D, D), lambda i: (0, 0)),
        pl.BlockSpec((D,), lambda i: (0,)),
        pl.BlockSpec((D, D), lambda i: (0, 0)),
        pl.BlockSpec((H, D), lambda i: (0, 0)),
    ],
    out_specs=[pl.BlockSpec((R, D), lambda i: (i, 0)),
               pl.BlockSpec((R, H), lambda i: (i, 0))],
    out_shape=[jax.ShapeDtypeStruct((NP, D), jnp.float32),
               jax.ShapeDtypeStruct((NP, H), jnp.float32)],
)


def _tc1_body(p_ref, cp_ref, y0_ref, bl1_ref, wr1_ref, h1_ref):
    m = _mean1(p_ref, _inv_cnt(cp_ref))
    h1_ref[...] = m + bl1_ref[...][None, :] + _dotT(y0_ref[...], wr1_ref[...])


_tc1 = pl.pallas_call(
    _tc1_body,
    grid=(G,),
    in_specs=[
        _vspec,
        _cspec,
        pl.BlockSpec((R, D), lambda i: (i, 0)),
        pl.BlockSpec((H,), lambda i: (0,)),
        pl.BlockSpec((H, D), lambda i: (0, 0)),
    ],
    out_specs=[pl.BlockSpec((R, H), lambda i: (i, 0))],
    out_shape=[jax.ShapeDtypeStruct((NP, H), jnp.float32)],
)


def _tc2_body(p_ref, cp_ref, h1_ref, wl2_ref, bl2_ref, wr2_ref,
              h2a_ref, h2b_ref):
    m = _mean1(p_ref, _inv_cnt(cp_ref))
    y = (_dotT(m, wl2_ref[...]) + bl2_ref[...][None, :]
         + _dotT(h1_ref[...], wr2_ref[...]))
    y = jnp.maximum(y, 0.0)
    h2a_ref[...] = y[:, :H]
    h2b_ref[...] = y[:, H:]


_tc2 = pl.pallas_call(
    _tc2_body,
    grid=(G,),
    in_specs=[
        _vspec,
        _cspec,
        pl.BlockSpec((R, H), lambda i: (i, 0)),
        pl.BlockSpec((D, H), lambda i: (0, 0)),
        pl.BlockSpec((D,), lambda i: (0,)),
        pl.BlockSpec((D, H), lambda i: (0, 0)),
    ],
    out_specs=[pl.BlockSpec((R, H), lambda i: (i, 0)),
               pl.BlockSpec((R, H), lambda i: (i, 0))],
    out_shape=[jax.ShapeDtypeStruct((NP, H), jnp.float32),
               jax.ShapeDtypeStruct((NP, H), jnp.float32)],
)


def _tc3_body(pa_ref, pb_ref, cp_ref, h2a_ref, h2b_ref, wl3_ref, bl3_ref,
              wr3_ref, o_ref):
    inv = _inv_cnt(cp_ref)
    ma = _mean1(pa_ref, inv)
    mb = _mean1(pb_ref, inv)
    o_ref[...] = (_dotT(ma, wl3_ref[:, :H]) + _dotT(mb, wl3_ref[:, H:])
                  + bl3_ref[...][None, :]
                  + _dotT(h2a_ref[...], wr3_ref[:, :H])
                  + _dotT(h2b_ref[...], wr3_ref[:, H:]))


_tc3 = pl.pallas_call(
    _tc3_body,
    grid=(G,),
    in_specs=[
        _vspec,
        _vspec,
        _cspec,
        pl.BlockSpec((R, H), lambda i: (i, 0)),
        pl.BlockSpec((R, H), lambda i: (i, 0)),
        pl.BlockSpec((D, D), lambda i: (0, 0)),
        pl.BlockSpec((D,), lambda i: (0,)),
        pl.BlockSpec((D, D), lambda i: (0, 0)),
    ],
    out_specs=[pl.BlockSpec((R, D), lambda i: (i, 0))],
    out_shape=[jax.ShapeDtypeStruct((NP, D), jnp.float32)],
)


def _first(res):
    return res[0] if isinstance(res, (list, tuple)) else res


def kernel(x, edge_index, Wl0, bl0, Wr0, Wl1, bl1, Wr1, Wl2, bl2, Wr2,
           Wl3, bl3, Wr3):
    pad = EP - E
    src3 = jnp.concatenate(
        [edge_index[0], jnp.zeros((pad,), jnp.int32)]).reshape(NW, K, CH)
    dst3 = jnp.concatenate(
        [edge_index[1], jnp.full((pad,), N, jnp.int32)]).reshape(NW, K, CH)
    x_p = jnp.pad(x, ((0, NP - N), (0, 0)))
    xa = x_p[:, :H]
    xb = x_p[:, H:]

    cp = _first(_sc_counts(dst3))
    p0a = _first(_sc64(xa, src3, dst3))
    p0b = _first(_sc64(xb, src3, dst3))
    y0, t1 = _tc0(p0a, p0b, cp, x_p, Wl0, bl0, Wr0, Wl1)
    p1 = _first(_sc64(t1, src3, dst3))
    h1 = _first(_tc1(p1, cp, y0, bl1, Wr1))
    p2 = _first(_sc64(h1, src3, dst3))
    h2a, h2b = _tc2(p2, cp, h1, Wl2, bl2, Wr2)
    p3a = _first(_sc64(h2a, src3, dst3))
    p3b = _first(_sc64(h2b, src3, dst3))
    out = _first(_tc3(p3a, p3b, cp, h2a, h2b, Wl3, bl3, Wr3))
    return out[:N]


# speedup vs baseline: 10.6382x; 1.0228x over previous
import functools

import jax
import jax.numpy as jnp
from jax import lax
from jax.experimental import pallas as pl
from jax.experimental.pallas import tpu as pltpu
from jax.experimental.pallas import tpu_sc as plsc

N = 10000
NP = 10240
D = 128
H = 64
E = 320000
CH = 128
NW = 32
K = 80
EP = NW * K * CH
GC = 4
NG = K // GC
RPT = NP // 16
ZR = 32


def _make_sc_agg64(srcw: int = H, col0: int = 0):
    F = H
    mesh = plsc.VectorSubcoreMesh(core_axis_name="c", subcore_axis_name="s")
    out_type = jax.ShapeDtypeStruct((2, NP, F), jnp.float32)
    scratch = [
        pltpu.VMEM((GC, CH), jnp.int32),
        pltpu.VMEM((GC, CH), jnp.int32),
        pltpu.VMEM((GC, CH), jnp.int32),
        pltpu.VMEM((GC, CH), jnp.int32),
        pltpu.VMEM((CH, F), jnp.float32),
        pltpu.VMEM((CH, F), jnp.float32),
        pltpu.VMEM((CH, F), jnp.float32),
        pltpu.VMEM((CH, F), jnp.float32),
        pltpu.VMEM((ZR, F), jnp.float32),
        pltpu.VMEM_SHARED((NP, F), jnp.float32),
        pltpu.VMEM_SHARED((NP, F), jnp.float32),
        pltpu.SemaphoreType.DMA,
        pltpu.SemaphoreType.DMA,
        pltpu.SemaphoreType.DMA,
        pltpu.SemaphoreType.DMA,
        pltpu.SemaphoreType.DMA,
        pltpu.SemaphoreType.DMA,
        pltpu.SemaphoreType.DMA,
        pltpu.SemaphoreType.DMA,
        pltpu.SemaphoreType.DMA,
    ]

    def body(h_hbm, src_hbm, dst_hbm, out_hbm, srcg0, dstg0, srcg1, dstg1,
             rows0, rows1, rows2, rows3, zb_v, feat_sh, acc_sh,
             gsem0, gsem1, gsem2, gsem3, ssem0, ssem1, ssem2, ssem3, zsem):
        c = lax.axis_index("c")
        s = lax.axis_index("s")
        w = c * 16 + s

        if srcw == F:
            src_slab = h_hbm.at[pl.ds(s * RPT, RPT)]
        else:
            src_slab = h_hbm.at[pl.ds(s * RPT, RPT), pl.ds(col0, F)]
        pltpu.async_copy(src_slab, feat_sh.at[pl.ds(s * RPT, RPT)], zsem)
        def zfill(i, carry):
            for j in range(F // 16):
                zb_v[i, pl.ds(j * 16, 16)] = jnp.zeros((16,), jnp.float32)
            return carry
        lax.fori_loop(0, ZR, zfill, 0)
        for t in range(RPT // ZR):
            pltpu.async_copy(zb_v, acc_sh.at[pl.ds(s * RPT + t * ZR, ZR)], zsem)
        pltpu.make_async_copy(src_slab,
                              feat_sh.at[pl.ds(s * RPT, RPT)], zsem).wait()
        for t in range(RPT // ZR):
            pltpu.make_async_copy(
                zb_v, acc_sh.at[pl.ds(s * RPT, ZR)], zsem).wait()
        plsc.subcore_barrier()

        idxbufs = ((srcg0, dstg0), (srcg1, dstg1))
        rowsb = (rows0, rows1, rows2, rows3)
        gsems = (gsem0, gsem1, gsem2, gsem3)
        ssems = (ssem0, ssem1, ssem2, ssem3)

        pltpu.sync_copy(src_hbm.at[w, pl.ds(0, GC)], srcg0)
        pltpu.sync_copy(dst_hbm.at[w, pl.ds(0, GC)], dstg0)
        pltpu.async_copy(feat_sh.at[srcg0.at[0]], rows0, gsem0)
        pltpu.async_copy(feat_sh.at[srcg0.at[1]], rows1, gsem1)

        def iter_body(i, carry):
            for gp in range(2):
                srcg, dstg = idxbufs[gp]
                nsrcg, ndstg = idxbufs[1 - gp]
                g = 2 * i + gp
                for j in range(GC):
                    k = g * GC + j
                    rows_v = rowsb[j]
                    nx = (j + 2) % 4
                    pltpu.make_async_copy(
                        feat_sh.at[srcg.at[j]], rows_v, gsems[j]).wait()
                    pltpu.async_copy(
                        rows_v, acc_sh.at[dstg.at[j]], ssems[j], add=True)

                    @pl.when(k >= 2)
                    def _():
                        if j < 2:
                            pltpu.make_async_copy(
                                rowsb[nx], acc_sh.at[ndstg.at[nx]],
                                ssems[nx]).wait()
                        else:
                            pltpu.make_async_copy(
                                rowsb[nx], acc_sh.at[dstg.at[nx]],
                                ssems[nx]).wait()

                    if j == 2:
                        @pl.when(g + 1 < NG)
                        def _():
                            pltpu.sync_copy(
                                src_hbm.at[w, pl.ds((g + 1) * GC, GC)], nsrcg)
                            pltpu.sync_copy(
                                dst_hbm.at[w, pl.ds((g + 1) * GC, GC)], ndstg)

                    @pl.when(k + 2 < K)
                    def _():
                        if j < GC - 2:
                            pltpu.async_copy(
                                feat_sh.at[srcg.at[j + 2]], rowsb[nx],
                                gsems[nx])
                        else:
                            pltpu.async_copy(
                                feat_sh.at[nsrcg.at[j + 2 - GC]], rowsb[nx],
                                gsems[nx])
            return carry
        lax.fori_loop(0, NG // 2, iter_body, 0)
        pltpu.make_async_copy(rows2, acc_sh.at[dstg1.at[2]], ssem2).wait()
        pltpu.make_async_copy(rows3, acc_sh.at[dstg1.at[3]], ssem3).wait()
        plsc.subcore_barrier()

        pltpu.sync_copy(acc_sh.at[pl.ds(s * RPT, RPT)],
                        out_hbm.at[c, pl.ds(s * RPT, RPT)])

    return pl.kernel(body, mesh=mesh, out_type=out_type, scratch_types=scratch,
                     compiler_params=pltpu.CompilerParams(
                         use_tc_tiling_on_sc=False))


def _make_sc_counts():
    mesh = plsc.VectorSubcoreMesh(core_axis_name="c", subcore_axis_name="s")
    out_type = jax.ShapeDtypeStruct((2, NP, 16), jnp.float32)
    scratch = [
        pltpu.VMEM((K, CH), jnp.int32),
        pltpu.VMEM((CH, 16), jnp.float32),
        pltpu.VMEM((ZR, 16), jnp.float32),
        pltpu.VMEM_SHARED((NP, 16), jnp.float32),
    ]

    def body(dst_hbm, cnt_hbm, dsts_v, ones_v, zc_v, cnt_sh):
        c = lax.axis_index("c")
        s = lax.axis_index("s")
        w = c * 16 + s

        def ofill(i, carry):
            ones_v[i, :] = jnp.full((16,), 1.0, jnp.float32)
            return carry
        lax.fori_loop(0, CH, ofill, 0)

        def zcfill(i, carry):
            zc_v[i, :] = jnp.zeros((16,), jnp.float32)
            return carry
        lax.fori_loop(0, ZR, zcfill, 0)
        for t in range(RPT // ZR):
            pltpu.sync_copy(zc_v, cnt_sh.at[pl.ds(s * RPT + t * ZR, ZR)])
        plsc.subcore_barrier()

        pltpu.sync_copy(dst_hbm.at[w], dsts_v)

        def chunk(k, carry):
            pltpu.sync_copy(ones_v, cnt_sh.at[dsts_v.at[k]], add=True)
            return carry
        lax.fori_loop(0, K, chunk, 0)
        plsc.subcore_barrier()

        pltpu.sync_copy(cnt_sh.at[pl.ds(s * RPT, RPT)],
                        cnt_hbm.at[c, pl.ds(s * RPT, RPT)])

    return pl.kernel(body, mesh=mesh, out_type=out_type, scratch_types=scratch,
                     compiler_params=pltpu.CompilerParams(
                         use_tc_tiling_on_sc=False))


_sc_counts = _make_sc_counts()
_sc64 = _make_sc_agg64()
_sc128a = _make_sc_agg64(D, 0)
_sc128b = _make_sc_agg64(D, H)


def _dotT(a, w):
    return lax.dot_general(a, w, (((1,), (1,)), ((), ())),
                           precision=lax.Precision.HIGHEST,
                           preferred_element_type=jnp.float32)


def _inv_cnt(cp_ref):
    cnt = cp_ref[0, :, 0:1] + cp_ref[1, :, 0:1]
    return 1.0 / jnp.maximum(cnt, 1.0)


def _mean1(p_ref, inv):
    return (p_ref[0] + p_ref[1]) * inv


R = 1024
G = NP // R

_vspec = pl.BlockSpec((2, R, H), lambda i: (0, i, 0))
_cspec = pl.BlockSpec((2, R, 16), lambda i: (0, i, 0))


def _tc0_body(pa_ref, pb_ref, cp_ref, x_ref, wl0_ref, bl0_ref, wr0_ref,
              wl1_ref, y0_ref, t1_ref):
    inv = _inv_cnt(cp_ref)
    ma = _mean1(pa_ref, inv)
    mb = _mean1(pb_ref, inv)
    y = (_dotT(ma, wl0_ref[:, :H]) + _dotT(mb, wl0_ref[:, H:])
         + bl0_ref[...][None, :] + _dotT(x_ref[...], wr0_ref[...]))
    y = jnp.maximum(y, 0.0)
    y0_ref[...] = y
    t1_ref[...] = _dotT(y, wl1_ref[...])


_tc0 = pl.pallas_call(
    _tc0_body,
    grid=(G,),
    in_specs=[
        _vspec,
        _vspec,
        _cspec,
        pl.BlockSpec((R, D), lambda i: (i, 0)),
        pl.BlockSpec((D, D), lambda i: (0, 0)),
        pl.BlockSpec((D,), lambda i: (0,)),
        pl.BlockSpec((D, D), lambda i: (0, 0)),
        pl.BlockSpec((H, D), lambda i: (0, 0)),
    ],
    out_specs=[pl.BlockSpec((R, D), lambda i: (i, 0)),
               pl.BlockSpec((R, H), lambda i: (i, 0))],
    out_shape=[jax.ShapeDtypeStruct((NP, D), jnp.float32),
               jax.ShapeDtypeStruct((NP, H), jnp.float32)],
)


def _tc1_body(p_ref, cp_ref, y0_ref, bl1_ref, wr1_ref, h1_ref):
    m = _mean1(p_ref, _inv_cnt(cp_ref))
    h1_ref[...] = m + bl1_ref[...][None, :] + _dotT(y0_ref[...], wr1_ref[...])


_tc1 = pl.pallas_call(
    _tc1_body,
    grid=(G,),
    in_specs=[
        _vspec,
        _cspec,
        pl.BlockSpec((R, D), lambda i: (i, 0)),
        pl.BlockSpec((H,), lambda i: (0,)),
        pl.BlockSpec((H, D), lambda i: (0, 0)),
    ],
    out_specs=[pl.BlockSpec((R, H), lambda i: (i, 0))],
    out_shape=[jax.ShapeDtypeStruct((NP, H), jnp.float32)],
)


def _tc2_body(p_ref, cp_ref, h1_ref, wl2_ref, bl2_ref, wr2_ref, h2_ref):
    m = _mean1(p_ref, _inv_cnt(cp_ref))
    y = (_dotT(m, wl2_ref[...]) + bl2_ref[...][None, :]
         + _dotT(h1_ref[...], wr2_ref[...]))
    h2_ref[...] = jnp.maximum(y, 0.0)


_tc2 = pl.pallas_call(
    _tc2_body,
    grid=(G,),
    in_specs=[
        _vspec,
        _cspec,
        pl.BlockSpec((R, H), lambda i: (i, 0)),
        pl.BlockSpec((D, H), lambda i: (0, 0)),
        pl.BlockSpec((D,), lambda i: (0,)),
        pl.BlockSpec((D, H), lambda i: (0, 0)),
    ],
    out_specs=[pl.BlockSpec((R, D), lambda i: (i, 0))],
    out_shape=[jax.ShapeDtypeStruct((NP, D), jnp.float32)],
)


def _tc3_body(pa_ref, pb_ref, cp_ref, h2_ref, wl3_ref, bl3_ref,
              wr3_ref, o_ref):
    inv = _inv_cnt(cp_ref)
    ma = _mean1(pa_ref, inv)
    mb = _mean1(pb_ref, inv)
    o_ref[...] = (_dotT(ma, wl3_ref[:, :H]) + _dotT(mb, wl3_ref[:, H:])
                  + bl3_ref[...][None, :]
                  + _dotT(h2_ref[...], wr3_ref[...]))


R3O = 1000
_tc3 = pl.pallas_call(
    _tc3_body,
    grid=(N // R3O,),
    in_specs=[
        pl.BlockSpec((2, R3O, H), lambda i: (0, i, 0)),
        pl.BlockSpec((2, R3O, H), lambda i: (0, i, 0)),
        pl.BlockSpec((2, R3O, 16), lambda i: (0, i, 0)),
        pl.BlockSpec((R3O, D), lambda i: (i, 0)),
        pl.BlockSpec((D, D), lambda i: (0, 0)),
        pl.BlockSpec((D,), lambda i: (0,)),
        pl.BlockSpec((D, D), lambda i: (0, 0)),
    ],
    out_specs=[pl.BlockSpec((R3O, D), lambda i: (i, 0))],
    out_shape=[jax.ShapeDtypeStruct((N, D), jnp.float32)],
)


def _first(res):
    return res[0] if isinstance(res, (list, tuple)) else res


def kernel(x, edge_index, Wl0, bl0, Wr0, Wl1, bl1, Wr1, Wl2, bl2, Wr2,
           Wl3, bl3, Wr3):
    pad = EP - E
    src3 = jnp.concatenate(
        [edge_index[0], jnp.zeros((pad,), jnp.int32)]).reshape(NW, K, CH)
    dst3 = jnp.concatenate(
        [edge_index[1], jnp.full((pad,), N, jnp.int32)]).reshape(NW, K, CH)
    x_p = jnp.pad(x, ((0, NP - N), (0, 0)))

    cp = _first(_sc_counts(dst3))
    p0a = _first(_sc128a(x_p, src3, dst3))
    p0b = _first(_sc128b(x_p, src3, dst3))
    y0, t1 = _tc0(p0a, p0b, cp, x_p, Wl0, bl0, Wr0, Wl1)
    p1 = _first(_sc64(t1, src3, dst3))
    h1 = _first(_tc1(p1, cp, y0, bl1, Wr1))
    p2 = _first(_sc64(h1, src3, dst3))
    h2 = _first(_tc2(p2, cp, h1, Wl2, bl2, Wr2))
    p3a = _first(_sc128a(h2, src3, dst3))
    p3b = _first(_sc128b(h2, src3, dst3))
    return _first(_tc3(p3a, p3b, cp, h2, Wl3, bl3, Wr3))


# speedup vs baseline: 10.7688x; 1.0123x over previous
import functools

import jax
import jax.numpy as jnp
from jax import lax
from jax.experimental import pallas as pl
from jax.experimental.pallas import tpu as pltpu
from jax.experimental.pallas import tpu_sc as plsc

N = 10000
NP = 10240
D = 128
H = 64
E = 320000
CH = 128
NW = 32
K = 80
EP = NW * K * CH
GC = 4
NG = K // GC
RPT = NP // 16
ZR = 32


def _make_sc_agg64(srcw: int = H, dual: bool = False):
    F = H
    mesh = plsc.VectorSubcoreMesh(core_axis_name="c", subcore_axis_name="s")
    if dual:
        out_type = [jax.ShapeDtypeStruct((2, NP, F), jnp.float32),
                    jax.ShapeDtypeStruct((2, NP, F), jnp.float32)]
    else:
        out_type = jax.ShapeDtypeStruct((2, NP, F), jnp.float32)
    scratch = [
        pltpu.VMEM((GC, CH), jnp.int32),
        pltpu.VMEM((GC, CH), jnp.int32),
        pltpu.VMEM((GC, CH), jnp.int32),
        pltpu.VMEM((GC, CH), jnp.int32),
        pltpu.VMEM((CH, F), jnp.float32),
        pltpu.VMEM((CH, F), jnp.float32),
        pltpu.VMEM((CH, F), jnp.float32),
        pltpu.VMEM((CH, F), jnp.float32),
        pltpu.VMEM((ZR, F), jnp.float32),
        pltpu.VMEM_SHARED((NP, F), jnp.float32),
        pltpu.VMEM_SHARED((NP, F), jnp.float32),
        pltpu.SemaphoreType.DMA,
        pltpu.SemaphoreType.DMA,
        pltpu.SemaphoreType.DMA,
        pltpu.SemaphoreType.DMA,
        pltpu.SemaphoreType.DMA,
        pltpu.SemaphoreType.DMA,
        pltpu.SemaphoreType.DMA,
        pltpu.SemaphoreType.DMA,
        pltpu.SemaphoreType.DMA,
    ]

    def body(h_hbm, src_hbm, dst_hbm, *rest):
        if dual:
            (outA_hbm, outB_hbm, srcg0, dstg0, srcg1, dstg1,
             rows0, rows1, rows2, rows3, zb_v, feat_sh, acc_sh,
             gsem0, gsem1, gsem2, gsem3, ssem0, ssem1, ssem2, ssem3,
             zsem) = rest
            passes = ((0, outA_hbm), (H, outB_hbm))
        else:
            (out_hbm, srcg0, dstg0, srcg1, dstg1,
             rows0, rows1, rows2, rows3, zb_v, feat_sh, acc_sh,
             gsem0, gsem1, gsem2, gsem3, ssem0, ssem1, ssem2, ssem3,
             zsem) = rest
            passes = ((0, out_hbm),)
        c = lax.axis_index("c")
        s = lax.axis_index("s")
        w = c * 16 + s

        idxbufs = ((srcg0, dstg0), (srcg1, dstg1))
        rowsb = (rows0, rows1, rows2, rows3)
        gsems = (gsem0, gsem1, gsem2, gsem3)
        ssems = (ssem0, ssem1, ssem2, ssem3)

        def zfill(i, carry):
            for j in range(F // 16):
                zb_v[i, pl.ds(j * 16, 16)] = jnp.zeros((16,), jnp.float32)
            return carry
        lax.fori_loop(0, ZR, zfill, 0)

        for col0, out_hbm in passes:
            if srcw == F:
                src_slab = h_hbm.at[pl.ds(s * RPT, RPT)]
            else:
                src_slab = h_hbm.at[pl.ds(s * RPT, RPT), pl.ds(col0, F)]
            pltpu.async_copy(src_slab, feat_sh.at[pl.ds(s * RPT, RPT)], zsem)
            for t in range(RPT // ZR):
                pltpu.async_copy(zb_v, acc_sh.at[pl.ds(s * RPT + t * ZR, ZR)],
                                 zsem)
            pltpu.make_async_copy(src_slab,
                                  feat_sh.at[pl.ds(s * RPT, RPT)], zsem).wait()
            for t in range(RPT // ZR):
                pltpu.make_async_copy(
                    zb_v, acc_sh.at[pl.ds(s * RPT, ZR)], zsem).wait()
            plsc.subcore_barrier()

            pltpu.sync_copy(src_hbm.at[w, pl.ds(0, GC)], srcg0)
            pltpu.sync_copy(dst_hbm.at[w, pl.ds(0, GC)], dstg0)
            pltpu.async_copy(feat_sh.at[srcg0.at[0]], rows0, gsem0)
            pltpu.async_copy(feat_sh.at[srcg0.at[1]], rows1, gsem1)

            def iter_body(i, carry):
                for gp in range(2):
                    srcg, dstg = idxbufs[gp]
                    nsrcg, ndstg = idxbufs[1 - gp]
                    g = 2 * i + gp
                    for j in range(GC):
                        k = g * GC + j
                        rows_v = rowsb[j]
                        nx = (j + 2) % 4
                        pltpu.make_async_copy(
                            feat_sh.at[srcg.at[j]], rows_v, gsems[j]).wait()
                        pltpu.async_copy(
                            rows_v, acc_sh.at[dstg.at[j]], ssems[j], add=True)

                        @pl.when(k >= 2)
                        def _():
                            if j < 2:
                                pltpu.make_async_copy(
                                    rowsb[nx], acc_sh.at[ndstg.at[nx]],
                                    ssems[nx]).wait()
                            else:
                                pltpu.make_async_copy(
                                    rowsb[nx], acc_sh.at[dstg.at[nx]],
                                    ssems[nx]).wait()

                        if j == 2:
                            @pl.when(g + 1 < NG)
                            def _():
                                pltpu.sync_copy(
                                    src_hbm.at[w, pl.ds((g + 1) * GC, GC)],
                                    nsrcg)
                                pltpu.sync_copy(
                                    dst_hbm.at[w, pl.ds((g + 1) * GC, GC)],
                                    ndstg)

                        @pl.when(k + 2 < K)
                        def _():
                            if j < GC - 2:
                                pltpu.async_copy(
                                    feat_sh.at[srcg.at[j + 2]], rowsb[nx],
                                    gsems[nx])
                            else:
                                pltpu.async_copy(
                                    feat_sh.at[nsrcg.at[j + 2 - GC]],
                                    rowsb[nx], gsems[nx])
                return carry
            lax.fori_loop(0, NG // 2, iter_body, 0)
            pltpu.make_async_copy(rows2, acc_sh.at[dstg1.at[2]], ssem2).wait()
            pltpu.make_async_copy(rows3, acc_sh.at[dstg1.at[3]], ssem3).wait()
            plsc.subcore_barrier()

            pltpu.sync_copy(acc_sh.at[pl.ds(s * RPT, RPT)],
                            out_hbm.at[c, pl.ds(s * RPT, RPT)])

    return pl.kernel(body, mesh=mesh, out_type=out_type, scratch_types=scratch,
                     compiler_params=pltpu.CompilerParams(
                         use_tc_tiling_on_sc=False))


def _make_sc_counts():
    mesh = plsc.VectorSubcoreMesh(core_axis_name="c", subcore_axis_name="s")
    out_type = jax.ShapeDtypeStruct((2, NP, 16), jnp.float32)
    scratch = [
        pltpu.VMEM((K, CH), jnp.int32),
        pltpu.VMEM((CH, 16), jnp.float32),
        pltpu.VMEM((ZR, 16), jnp.float32),
        pltpu.VMEM_SHARED((NP, 16), jnp.float32),
    ]

    def body(dst_hbm, cnt_hbm, dsts_v, ones_v, zc_v, cnt_sh):
        c = lax.axis_index("c")
        s = lax.axis_index("s")
        w = c * 16 + s

        def ofill(i, carry):
            ones_v[i, :] = jnp.full((16,), 1.0, jnp.float32)
            return carry
        lax.fori_loop(0, CH, ofill, 0)

        def zcfill(i, carry):
            zc_v[i, :] = jnp.zeros((16,), jnp.float32)
            return carry
        lax.fori_loop(0, ZR, zcfill, 0)
        for t in range(RPT // ZR):
            pltpu.sync_copy(zc_v, cnt_sh.at[pl.ds(s * RPT + t * ZR, ZR)])
        plsc.subcore_barrier()

        pltpu.sync_copy(dst_hbm.at[w], dsts_v)

        def chunk(k, carry):
            pltpu.sync_copy(ones_v, cnt_sh.at[dsts_v.at[k]], add=True)
            return carry
        lax.fori_loop(0, K, chunk, 0)
        plsc.subcore_barrier()

        pltpu.sync_copy(cnt_sh.at[pl.ds(s * RPT, RPT)],
                        cnt_hbm.at[c, pl.ds(s * RPT, RPT)])

    return pl.kernel(body, mesh=mesh, out_type=out_type, scratch_types=scratch,
                     compiler_params=pltpu.CompilerParams(
                         use_tc_tiling_on_sc=False))


_sc_counts = _make_sc_counts()
_sc64 = _make_sc_agg64()
_sc128d = _make_sc_agg64(D, dual=True)


def _dotT(a, w):
    return lax.dot_general(a, w, (((1,), (1,)), ((), ())),
                           precision=lax.Precision.HIGHEST,
                           preferred_element_type=jnp.float32)


def _inv_cnt(cp_ref):
    cnt = cp_ref[0, :, 0:1] + cp_ref[1, :, 0:1]
    return 1.0 / jnp.maximum(cnt, 1.0)


def _mean1(p_ref, inv):
    return (p_ref[0] + p_ref[1]) * inv


R = 1024
G = NP // R

_vspec = pl.BlockSpec((2, R, H), lambda i: (0, i, 0))
_cspec = pl.BlockSpec((2, R, 16), lambda i: (0, i, 0))


def _tc0_body(pa_ref, pb_ref, cp_ref, x_ref, wl0_ref, bl0_ref, wr0_ref,
              wl1_ref, y0_ref, t1_ref):
    inv = _inv_cnt(cp_ref)
    ma = _mean1(pa_ref, inv)
    mb = _mean1(pb_ref, inv)
    y = (_dotT(ma, wl0_ref[:, :H]) + _dotT(mb, wl0_ref[:, H:])
         + bl0_ref[...][None, :] + _dotT(x_ref[...], wr0_ref[...]))
    y = jnp.maximum(y, 0.0)
    y0_ref[...] = y
    t1_ref[...] = _dotT(y, wl1_ref[...])


_tc0 = pl.pallas_call(
    _tc0_body,
    grid=(G,),
    in_specs=[
        _vspec,
        _vspec,
        _cspec,
        pl.BlockSpec((R, D), lambda i: (i, 0)),
        pl.BlockSpec((D, D), lambda i: (0, 0)),
        pl.BlockSpec((D,), lambda i: (0,)),
        pl.BlockSpec((D, D), lambda i: (0, 0)),
        pl.BlockSpec((H, D), lambda i: (0, 0)),
    ],
    out_specs=[pl.BlockSpec((R, D), lambda i: (i, 0)),
               pl.BlockSpec((R, H), lambda i: (i, 0))],
    out_shape=[jax.ShapeDtypeStruct((NP, D), jnp.float32),
               jax.ShapeDtypeStruct((NP, H), jnp.float32)],
)


def _tc1_body(p_ref, cp_ref, y0_ref, bl1_ref, wr1_ref, h1_ref):
    m = _mean1(p_ref, _inv_cnt(cp_ref))
    h1_ref[...] = m + bl1_ref[...][None, :] + _dotT(y0_ref[...], wr1_ref[...])


_tc1 = pl.pallas_call(
    _tc1_body,
    grid=(G,),
    in_specs=[
        _vspec,
        _cspec,
        pl.BlockSpec((R, D), lambda i: (i, 0)),
        pl.BlockSpec((H,), lambda i: (0,)),
        pl.BlockSpec((H, D), lambda i: (0, 0)),
    ],
    out_specs=[pl.BlockSpec((R, H), lambda i: (i, 0))],
    out_shape=[jax.ShapeDtypeStruct((NP, H), jnp.float32)],
)


def _tc2_body(p_ref, cp_ref, h1_ref, wl2_ref, bl2_ref, wr2_ref, h2_ref):
    m = _mean1(p_ref, _inv_cnt(cp_ref))
    y = (_dotT(m, wl2_ref[...]) + bl2_ref[...][None, :]
         + _dotT(h1_ref[...], wr2_ref[...]))
    h2_ref[...] = jnp.maximum(y, 0.0)


_tc2 = pl.pallas_call(
    _tc2_body,
    grid=(G,),
    in_specs=[
        _vspec,
        _cspec,
        pl.BlockSpec((R, H), lambda i: (i, 0)),
        pl.BlockSpec((D, H), lambda i: (0, 0)),
        pl.BlockSpec((D,), lambda i: (0,)),
        pl.BlockSpec((D, H), lambda i: (0, 0)),
    ],
    out_specs=[pl.BlockSpec((R, D), lambda i: (i, 0))],
    out_shape=[jax.ShapeDtypeStruct((NP, D), jnp.float32)],
)


def _tc3_body(pa_ref, pb_ref, cp_ref, h2_ref, wl3_ref, bl3_ref,
              wr3_ref, o_ref):
    inv = _inv_cnt(cp_ref)
    ma = _mean1(pa_ref, inv)
    mb = _mean1(pb_ref, inv)
    o_ref[...] = (_dotT(ma, wl3_ref[:, :H]) + _dotT(mb, wl3_ref[:, H:])
                  + bl3_ref[...][None, :]
                  + _dotT(h2_ref[...], wr3_ref[...]))


R3O = 1000
_tc3 = pl.pallas_call(
    _tc3_body,
    grid=(N // R3O,),
    in_specs=[
        pl.BlockSpec((2, R3O, H), lambda i: (0, i, 0)),
        pl.BlockSpec((2, R3O, H), lambda i: (0, i, 0)),
        pl.BlockSpec((2, R3O, 16), lambda i: (0, i, 0)),
        pl.BlockSpec((R3O, D), lambda i: (i, 0)),
        pl.BlockSpec((D, D), lambda i: (0, 0)),
        pl.BlockSpec((D,), lambda i: (0,)),
        pl.BlockSpec((D, D), lambda i: (0, 0)),
    ],
    out_specs=[pl.BlockSpec((R3O, D), lambda i: (i, 0))],
    out_shape=[jax.ShapeDtypeStruct((N, D), jnp.float32)],
)


def _first(res):
    return res[0] if isinstance(res, (list, tuple)) else res


def kernel(x, edge_index, Wl0, bl0, Wr0, Wl1, bl1, Wr1, Wl2, bl2, Wr2,
           Wl3, bl3, Wr3):
    pad = EP - E
    src3 = jnp.concatenate(
        [edge_index[0], jnp.zeros((pad,), jnp.int32)]).reshape(NW, K, CH)
    dst3 = jnp.concatenate(
        [edge_index[1], jnp.full((pad,), N, jnp.int32)]).reshape(NW, K, CH)
    x_p = jnp.pad(x, ((0, NP - N), (0, 0)))

    cp = _first(_sc_counts(dst3))
    p0a, p0b = _sc128d(x_p, src3, dst3)
    y0, t1 = _tc0(p0a, p0b, cp, x_p, Wl0, bl0, Wr0, Wl1)
    p1 = _first(_sc64(t1, src3, dst3))
    h1 = _first(_tc1(p1, cp, y0, bl1, Wr1))
    p2 = _first(_sc64(h1, src3, dst3))
    h2 = _first(_tc2(p2, cp, h1, Wl2, bl2, Wr2))
    p3a, p3b = _sc128d(h2, src3, dst3)
    return _first(_tc3(p3a, p3b, cp, h2, Wl3, bl3, Wr3))


# speedup vs baseline: 11.0162x; 1.0230x over previous
import functools

import jax
import jax.numpy as jnp
from jax import lax
from jax.experimental import pallas as pl
from jax.experimental.pallas import tpu as pltpu
from jax.experimental.pallas import tpu_sc as plsc

N = 10000
NP = 10240
D = 128
H = 64
E = 320000
CH = 128
NW = 32
K = 80
EP = NW * K * CH
GC = 4
NG = K // GC
RPT = NP // 16
ZR = 32


def _make_sc_agg64(srcw: int = H, dual: bool = False):
    F = H
    mesh = plsc.VectorSubcoreMesh(core_axis_name="c", subcore_axis_name="s")
    if dual:
        out_type = [jax.ShapeDtypeStruct((2, NP, F), jnp.float32),
                    jax.ShapeDtypeStruct((2, NP, F), jnp.float32)]
    else:
        out_type = jax.ShapeDtypeStruct((2, NP, F), jnp.float32)
    scratch = [
        pltpu.VMEM((GC, CH), jnp.int32),
        pltpu.VMEM((GC, CH), jnp.int32),
        pltpu.VMEM((GC, CH), jnp.int32),
        pltpu.VMEM((GC, CH), jnp.int32),
        pltpu.VMEM((CH, F), jnp.float32),
        pltpu.VMEM((CH, F), jnp.float32),
        pltpu.VMEM((CH, F), jnp.float32),
        pltpu.VMEM((CH, F), jnp.float32),
        pltpu.VMEM((ZR, F), jnp.float32),
        pltpu.VMEM_SHARED((NP, F), jnp.float32),
        pltpu.VMEM_SHARED((NP, F), jnp.float32),
        pltpu.SemaphoreType.DMA,
        pltpu.SemaphoreType.DMA,
        pltpu.SemaphoreType.DMA,
        pltpu.SemaphoreType.DMA,
        pltpu.SemaphoreType.DMA,
        pltpu.SemaphoreType.DMA,
        pltpu.SemaphoreType.DMA,
        pltpu.SemaphoreType.DMA,
        pltpu.SemaphoreType.DMA,
    ]

    def body(h_hbm, src_hbm, dst_hbm, *rest):
        if dual:
            (outA_hbm, outB_hbm, srcg0, dstg0, srcg1, dstg1,
             rows0, rows1, rows2, rows3, zb_v, feat_sh, acc_sh,
             gsem0, gsem1, gsem2, gsem3, ssem0, ssem1, ssem2, ssem3,
             zsem) = rest
            passes = ((0, outA_hbm), (H, outB_hbm))
        else:
            (out_hbm, srcg0, dstg0, srcg1, dstg1,
             rows0, rows1, rows2, rows3, zb_v, feat_sh, acc_sh,
             gsem0, gsem1, gsem2, gsem3, ssem0, ssem1, ssem2, ssem3,
             zsem) = rest
            passes = ((0, out_hbm),)
        c = lax.axis_index("c")
        s = lax.axis_index("s")
        w = c * 16 + s

        idxbufs = ((srcg0, dstg0), (srcg1, dstg1))
        rowsb = (rows0, rows1, rows2, rows3)
        gsems = (gsem0, gsem1, gsem2, gsem3)
        ssems = (ssem0, ssem1, ssem2, ssem3)

        def zfill(i, carry):
            for j in range(F // 16):
                zb_v[i, pl.ds(j * 16, 16)] = jnp.zeros((16,), jnp.float32)
            return carry
        lax.fori_loop(0, ZR, zfill, 0)

        for col0, out_hbm in passes:
            if srcw == F:
                src_slab = h_hbm.at[pl.ds(s * RPT, RPT)]
            else:
                src_slab = h_hbm.at[pl.ds(s * RPT, RPT), pl.ds(col0, F)]
            pltpu.async_copy(src_slab, feat_sh.at[pl.ds(s * RPT, RPT)], zsem)
            for t in range(RPT // ZR):
                pltpu.async_copy(zb_v, acc_sh.at[pl.ds(s * RPT + t * ZR, ZR)],
                                 zsem)
            pltpu.make_async_copy(src_slab,
                                  feat_sh.at[pl.ds(s * RPT, RPT)], zsem).wait()
            for t in range(RPT // ZR):
                pltpu.make_async_copy(
                    zb_v, acc_sh.at[pl.ds(s * RPT, ZR)], zsem).wait()
            plsc.subcore_barrier()

            pltpu.sync_copy(src_hbm.at[w, pl.ds(0, GC)], srcg0)
            pltpu.sync_copy(dst_hbm.at[w, pl.ds(0, GC)], dstg0)
            pltpu.async_copy(feat_sh.at[srcg0.at[0]], rows0, gsem0)
            pltpu.async_copy(feat_sh.at[srcg0.at[1]], rows1, gsem1)

            def iter_body(i, carry):
                for gp in range(2):
                    srcg, dstg = idxbufs[gp]
                    nsrcg, ndstg = idxbufs[1 - gp]
                    g = 2 * i + gp
                    for j in range(GC):
                        k = g * GC + j
                        rows_v = rowsb[j]
                        nx = (j + 2) % 4
                        pltpu.make_async_copy(
                            feat_sh.at[srcg.at[j]], rows_v, gsems[j]).wait()
                        pltpu.async_copy(
                            rows_v, acc_sh.at[dstg.at[j]], ssems[j], add=True)

                        @pl.when(k >= 2)
                        def _():
                            if j < 2:
                                pltpu.make_async_copy(
                                    rowsb[nx], acc_sh.at[ndstg.at[nx]],
                                    ssems[nx]).wait()
                            else:
                                pltpu.make_async_copy(
                                    rowsb[nx], acc_sh.at[dstg.at[nx]],
                                    ssems[nx]).wait()

                        if j == 2:
                            @pl.when(g + 1 < NG)
                            def _():
                                pltpu.sync_copy(
                                    src_hbm.at[w, pl.ds((g + 1) * GC, GC)],
                                    nsrcg)
                                pltpu.sync_copy(
                                    dst_hbm.at[w, pl.ds((g + 1) * GC, GC)],
                                    ndstg)

                        @pl.when(k + 2 < K)
                        def _():
                            if j < GC - 2:
                                pltpu.async_copy(
                                    feat_sh.at[srcg.at[j + 2]], rowsb[nx],
                                    gsems[nx])
                            else:
                                pltpu.async_copy(
                                    feat_sh.at[nsrcg.at[j + 2 - GC]],
                                    rowsb[nx], gsems[nx])
                return carry
            lax.fori_loop(0, NG // 2, iter_body, 0)
            pltpu.make_async_copy(rows2, acc_sh.at[dstg1.at[2]], ssem2).wait()
            pltpu.make_async_copy(rows3, acc_sh.at[dstg1.at[3]], ssem3).wait()
            plsc.subcore_barrier()

            pltpu.sync_copy(acc_sh.at[pl.ds(s * RPT, RPT)],
                            out_hbm.at[c, pl.ds(s * RPT, RPT)])

    return pl.kernel(body, mesh=mesh, out_type=out_type, scratch_types=scratch,
                     compiler_params=pltpu.CompilerParams(
                         use_tc_tiling_on_sc=False))


def _make_sc_counts():
    mesh = plsc.VectorSubcoreMesh(core_axis_name="c", subcore_axis_name="s")
    out_type = jax.ShapeDtypeStruct((2, NP, 16), jnp.float32)
    scratch = [
        pltpu.VMEM((K, CH), jnp.int32),
        pltpu.VMEM((CH, 16), jnp.float32),
        pltpu.VMEM((ZR, 16), jnp.float32),
        pltpu.VMEM_SHARED((NP, 16), jnp.float32),
    ]

    def body(dst_hbm, cnt_hbm, dsts_v, ones_v, zc_v, cnt_sh):
        c = lax.axis_index("c")
        s = lax.axis_index("s")
        w = c * 16 + s

        def ofill(i, carry):
            ones_v[i, :] = jnp.full((16,), 1.0, jnp.float32)
            return carry
        lax.fori_loop(0, CH, ofill, 0)

        def zcfill(i, carry):
            zc_v[i, :] = jnp.zeros((16,), jnp.float32)
            return carry
        lax.fori_loop(0, ZR, zcfill, 0)
        for t in range(RPT // ZR):
            pltpu.sync_copy(zc_v, cnt_sh.at[pl.ds(s * RPT + t * ZR, ZR)])
        plsc.subcore_barrier()

        pltpu.sync_copy(dst_hbm.at[w], dsts_v)

        def chunk(k, carry):
            pltpu.sync_copy(ones_v, cnt_sh.at[dsts_v.at[k]], add=True)
            return carry
        lax.fori_loop(0, K, chunk, 0)
        plsc.subcore_barrier()

        pltpu.sync_copy(cnt_sh.at[pl.ds(s * RPT, RPT)],
                        cnt_hbm.at[c, pl.ds(s * RPT, RPT)])

    return pl.kernel(body, mesh=mesh, out_type=out_type, scratch_types=scratch,
                     compiler_params=pltpu.CompilerParams(
                         use_tc_tiling_on_sc=False))


_sc_counts = _make_sc_counts()
_sc64 = _make_sc_agg64()
_sc128d = _make_sc_agg64(D, dual=True)


def _dotT(a, w):
    return lax.dot_general(a, w, (((1,), (1,)), ((), ())),
                           precision=lax.Precision.HIGHEST,
                           preferred_element_type=jnp.float32)


def _inv_cnt(cp_ref):
    cnt = cp_ref[0, :, 0:1] + cp_ref[1, :, 0:1]
    return 1.0 / jnp.maximum(cnt, 1.0)


def _mean1(p_ref, inv):
    return (p_ref[0] + p_ref[1]) * inv


R = 1024
G = NP // R

_vspec = pl.BlockSpec((2, R, H), lambda i: (0, i, 0))
_cspec = pl.BlockSpec((2, R, 16), lambda i: (0, i, 0))


def _make_tc_mm(din, dout):
    def mm_body(x_ref, w_ref, o_ref):
        o_ref[...] = _dotT(x_ref[...], w_ref[...])

    return pl.pallas_call(
        mm_body,
        grid=(G,),
        in_specs=[pl.BlockSpec((R, din), lambda i: (i, 0)),
                  pl.BlockSpec((dout, din), lambda i: (0, 0))],
        out_specs=[pl.BlockSpec((R, dout), lambda i: (i, 0))],
        out_shape=[jax.ShapeDtypeStruct((NP, dout), jnp.float32)],
    )


_mm_d_d = _make_tc_mm(D, D)
_mm_d_h = _make_tc_mm(D, H)
_mm_h_d = _make_tc_mm(H, D)


def _tc0_body(pa_ref, pb_ref, cp_ref, xw_ref, wl0_ref, bl0_ref,
              wl1_ref, y0_ref, t1_ref):
    inv = _inv_cnt(cp_ref)
    ma = _mean1(pa_ref, inv)
    mb = _mean1(pb_ref, inv)
    y = (_dotT(ma, wl0_ref[:, :H]) + _dotT(mb, wl0_ref[:, H:])
         + bl0_ref[...][None, :] + xw_ref[...])
    y = jnp.maximum(y, 0.0)
    y0_ref[...] = y
    t1_ref[...] = _dotT(y, wl1_ref[...])


_tc0 = pl.pallas_call(
    _tc0_body,
    grid=(G,),
    in_specs=[
        _vspec,
        _vspec,
        _cspec,
        pl.BlockSpec((R, D), lambda i: (i, 0)),
        pl.BlockSpec((D, D), lambda i: (0, 0)),
        pl.BlockSpec((D,), lambda i: (0,)),
        pl.BlockSpec((H, D), lambda i: (0, 0)),
    ],
    out_specs=[pl.BlockSpec((R, D), lambda i: (i, 0)),
               pl.BlockSpec((R, H), lambda i: (i, 0))],
    out_shape=[jax.ShapeDtypeStruct((NP, D), jnp.float32),
               jax.ShapeDtypeStruct((NP, H), jnp.float32)],
)


def _tc1_body(p_ref, cp_ref, yw_ref, bl1_ref, h1_ref):
    m = _mean1(p_ref, _inv_cnt(cp_ref))
    h1_ref[...] = m + bl1_ref[...][None, :] + yw_ref[...]


_tc1 = pl.pallas_call(
    _tc1_body,
    grid=(G,),
    in_specs=[
        _vspec,
        _cspec,
        pl.BlockSpec((R, H), lambda i: (i, 0)),
        pl.BlockSpec((H,), lambda i: (0,)),
    ],
    out_specs=[pl.BlockSpec((R, H), lambda i: (i, 0))],
    out_shape=[jax.ShapeDtypeStruct((NP, H), jnp.float32)],
)


def _tc2_body(p_ref, cp_ref, hw_ref, wl2_ref, bl2_ref, h2_ref):
    m = _mean1(p_ref, _inv_cnt(cp_ref))
    y = _dotT(m, wl2_ref[...]) + bl2_ref[...][None, :] + hw_ref[...]
    h2_ref[...] = jnp.maximum(y, 0.0)


_tc2 = pl.pallas_call(
    _tc2_body,
    grid=(G,),
    in_specs=[
        _vspec,
        _cspec,
        pl.BlockSpec((R, D), lambda i: (i, 0)),
        pl.BlockSpec((D, H), lambda i: (0, 0)),
        pl.BlockSpec((D,), lambda i: (0,)),
    ],
    out_specs=[pl.BlockSpec((R, D), lambda i: (i, 0))],
    out_shape=[jax.ShapeDtypeStruct((NP, D), jnp.float32)],
)


def _tc3_body(pa_ref, pb_ref, cp_ref, h2w_ref, wl3_ref, bl3_ref, o_ref):
    inv = _inv_cnt(cp_ref)
    ma = _mean1(pa_ref, inv)
    mb = _mean1(pb_ref, inv)
    o_ref[...] = (_dotT(ma, wl3_ref[:, :H]) + _dotT(mb, wl3_ref[:, H:])
                  + bl3_ref[...][None, :] + h2w_ref[...])


R3O = 1000
_tc3 = pl.pallas_call(
    _tc3_body,
    grid=(N // R3O,),
    in_specs=[
        pl.BlockSpec((2, R3O, H), lambda i: (0, i, 0)),
        pl.BlockSpec((2, R3O, H), lambda i: (0, i, 0)),
        pl.BlockSpec((2, R3O, 16), lambda i: (0, i, 0)),
        pl.BlockSpec((R3O, D), lambda i: (i, 0)),
        pl.BlockSpec((D, D), lambda i: (0, 0)),
        pl.BlockSpec((D,), lambda i: (0,)),
    ],
    out_specs=[pl.BlockSpec((R3O, D), lambda i: (i, 0))],
    out_shape=[jax.ShapeDtypeStruct((N, D), jnp.float32)],
)


def _first(res):
    return res[0] if isinstance(res, (list, tuple)) else res


def kernel(x, edge_index, Wl0, bl0, Wr0, Wl1, bl1, Wr1, Wl2, bl2, Wr2,
           Wl3, bl3, Wr3):
    pad = EP - E
    src3 = jnp.concatenate(
        [edge_index[0], jnp.zeros((pad,), jnp.int32)]).reshape(NW, K, CH)
    dst3 = jnp.concatenate(
        [edge_index[1], jnp.full((pad,), N, jnp.int32)]).reshape(NW, K, CH)
    x_p = jnp.pad(x, ((0, NP - N), (0, 0)))

    cp = _first(_sc_counts(dst3))
    xw = _first(_mm_d_d(x_p, Wr0))
    p0a, p0b = _sc128d(x_p, src3, dst3)
    y0, t1 = _tc0(p0a, p0b, cp, xw, Wl0, bl0, Wl1)
    yw = _first(_mm_d_h(y0, Wr1))
    p1 = _first(_sc64(t1, src3, dst3))
    h1 = _first(_tc1(p1, cp, yw, bl1))
    hw = _first(_mm_h_d(h1, Wr2))
    p2 = _first(_sc64(h1, src3, dst3))
    h2 = _first(_tc2(p2, cp, hw, Wl2, bl2))
    h2w = _first(_mm_d_d(h2, Wr3))
    p3a, p3b = _sc128d(h2, src3, dst3)
    return _first(_tc3(p3a, p3b, cp, h2w, Wl3, bl3))


# speedup vs baseline: 11.0482x; 1.0029x over previous
import functools

import jax
import jax.numpy as jnp
from jax import lax
from jax.experimental import pallas as pl
from jax.experimental.pallas import tpu as pltpu
from jax.experimental.pallas import tpu_sc as plsc

N = 10000
NP = 10240
D = 128
H = 64
E = 320000
CH = 128
NW = 32
K = 80
EP = NW * K * CH
GC = 4
NG = K // GC
RPT = NP // 16
ZR = 32


def _make_sc_agg64(srcw: int = H, dual: bool = False, hist: bool = False):
    F = H
    mesh = plsc.VectorSubcoreMesh(core_axis_name="c", subcore_axis_name="s")
    if dual:
        out_type = [jax.ShapeDtypeStruct((2, NP, F), jnp.float32),
                    jax.ShapeDtypeStruct((2, NP, F), jnp.float32)]
        if hist:
            out_type.append(jax.ShapeDtypeStruct((NW, NP), jnp.float32))
    else:
        out_type = jax.ShapeDtypeStruct((2, NP, F), jnp.float32)
    scratch = [
        pltpu.VMEM((GC, CH), jnp.int32),
        pltpu.VMEM((GC, CH), jnp.int32),
        pltpu.VMEM((GC, CH), jnp.int32),
        pltpu.VMEM((GC, CH), jnp.int32),
        pltpu.VMEM((CH, F), jnp.float32),
        pltpu.VMEM((CH, F), jnp.float32),
        pltpu.VMEM((CH, F), jnp.float32),
        pltpu.VMEM((CH, F), jnp.float32),
        pltpu.VMEM((ZR, F), jnp.float32),
        pltpu.VMEM_SHARED((NP, F), jnp.float32),
        pltpu.VMEM_SHARED((NP, F), jnp.float32),
        pltpu.SemaphoreType.DMA,
        pltpu.SemaphoreType.DMA,
        pltpu.SemaphoreType.DMA,
        pltpu.SemaphoreType.DMA,
        pltpu.SemaphoreType.DMA,
        pltpu.SemaphoreType.DMA,
        pltpu.SemaphoreType.DMA,
        pltpu.SemaphoreType.DMA,
        pltpu.SemaphoreType.DMA,
    ]
    if hist:
        scratch = scratch + [pltpu.VMEM((NP,), jnp.float32),
                             pltpu.VMEM((CH,), jnp.int32)]

    def body(h_hbm, src_hbm, dst_hbm, *rest):
        hist_hbm = hist_v = None
        if dual and hist:
            (outA_hbm, outB_hbm, hist_hbm, srcg0, dstg0, srcg1, dstg1,
             rows0, rows1, rows2, rows3, zb_v, feat_sh, acc_sh,
             gsem0, gsem1, gsem2, gsem3, ssem0, ssem1, ssem2, ssem3,
             zsem, hist_v, hbuf_v) = rest
            passes = ((0, outA_hbm), (H, outB_hbm))
        elif dual:
            (outA_hbm, outB_hbm, srcg0, dstg0, srcg1, dstg1,
             rows0, rows1, rows2, rows3, zb_v, feat_sh, acc_sh,
             gsem0, gsem1, gsem2, gsem3, ssem0, ssem1, ssem2, ssem3,
             zsem) = rest
            passes = ((0, outA_hbm), (H, outB_hbm))
        else:
            (out_hbm, srcg0, dstg0, srcg1, dstg1,
             rows0, rows1, rows2, rows3, zb_v, feat_sh, acc_sh,
             gsem0, gsem1, gsem2, gsem3, ssem0, ssem1, ssem2, ssem3,
             zsem) = rest
            passes = ((0, out_hbm),)
        c = lax.axis_index("c")
        s = lax.axis_index("s")
        w = c * 16 + s

        idxbufs = ((srcg0, dstg0), (srcg1, dstg1))
        rowsb = (rows0, rows1, rows2, rows3)
        gsems = (gsem0, gsem1, gsem2, gsem3)
        ssems = (ssem0, ssem1, ssem2, ssem3)

        def zfill(i, carry):
            for j in range(F // 16):
                zb_v[i, pl.ds(j * 16, 16)] = jnp.zeros((16,), jnp.float32)
            return carry
        lax.fori_loop(0, ZR, zfill, 0)

        if hist:
            def hzero(i, carry):
                hist_v[pl.ds(i * 16, 16)] = jnp.zeros((16,), jnp.float32)
                return carry
            lax.fori_loop(0, NP // 16, hzero, 0)
            ones16 = jnp.full((16,), 1.0, jnp.float32)

            def hgrp(q, carry):
                pltpu.sync_copy(dst_hbm.at[w, q], hbuf_v)

                def hstep(t, carry2):
                    idx16 = hbuf_v[pl.ds(t * 16, 16)]
                    plsc.addupdate_scatter(hist_v, [idx16], ones16)
                    return carry2
                lax.fori_loop(0, CH // 16, hstep, 0)
                return carry
            lax.fori_loop(0, K, hgrp, 0)
            pltpu.sync_copy(hist_v, hist_hbm.at[w])

        for col0, out_hbm in passes:
            if srcw == F:
                src_slab = h_hbm.at[pl.ds(s * RPT, RPT)]
            else:
                src_slab = h_hbm.at[pl.ds(s * RPT, RPT), pl.ds(col0, F)]
            pltpu.async_copy(src_slab, feat_sh.at[pl.ds(s * RPT, RPT)], zsem)
            for t in range(RPT // ZR):
                pltpu.async_copy(zb_v, acc_sh.at[pl.ds(s * RPT + t * ZR, ZR)],
                                 zsem)
            pltpu.make_async_copy(src_slab,
                                  feat_sh.at[pl.ds(s * RPT, RPT)], zsem).wait()
            for t in range(RPT // ZR):
                pltpu.make_async_copy(
                    zb_v, acc_sh.at[pl.ds(s * RPT, ZR)], zsem).wait()
            plsc.subcore_barrier()

            pltpu.sync_copy(src_hbm.at[w, pl.ds(0, GC)], srcg0)
            pltpu.sync_copy(dst_hbm.at[w, pl.ds(0, GC)], dstg0)
            pltpu.async_copy(feat_sh.at[srcg0.at[0]], rows0, gsem0)
            pltpu.async_copy(feat_sh.at[srcg0.at[1]], rows1, gsem1)

            def iter_body(i, carry):
                for gp in range(2):
                    srcg, dstg = idxbufs[gp]
                    nsrcg, ndstg = idxbufs[1 - gp]
                    g = 2 * i + gp
                    for j in range(GC):
                        k = g * GC + j
                        rows_v = rowsb[j]
                        nx = (j + 2) % 4
                        pltpu.make_async_copy(
                            feat_sh.at[srcg.at[j]], rows_v, gsems[j]).wait()
                        pltpu.async_copy(
                            rows_v, acc_sh.at[dstg.at[j]], ssems[j], add=True)

                        @pl.when(k >= 2)
                        def _():
                            if j < 2:
                                pltpu.make_async_copy(
                                    rowsb[nx], acc_sh.at[ndstg.at[nx]],
                                    ssems[nx]).wait()
                            else:
                                pltpu.make_async_copy(
                                    rowsb[nx], acc_sh.at[dstg.at[nx]],
                                    ssems[nx]).wait()

                        if j == 2:
                            @pl.when(g + 1 < NG)
                            def _():
                                pltpu.sync_copy(
                                    src_hbm.at[w, pl.ds((g + 1) * GC, GC)],
                                    nsrcg)
                                pltpu.sync_copy(
                                    dst_hbm.at[w, pl.ds((g + 1) * GC, GC)],
                                    ndstg)

                        @pl.when(k + 2 < K)
                        def _():
                            if j < GC - 2:
                                pltpu.async_copy(
                                    feat_sh.at[srcg.at[j + 2]], rowsb[nx],
                                    gsems[nx])
                            else:
                                pltpu.async_copy(
                                    feat_sh.at[nsrcg.at[j + 2 - GC]],
                                    rowsb[nx], gsems[nx])
                return carry
            lax.fori_loop(0, NG // 2, iter_body, 0)
            pltpu.make_async_copy(rows2, acc_sh.at[dstg1.at[2]], ssem2).wait()
            pltpu.make_async_copy(rows3, acc_sh.at[dstg1.at[3]], ssem3).wait()
            plsc.subcore_barrier()

            pltpu.sync_copy(acc_sh.at[pl.ds(s * RPT, RPT)],
                            out_hbm.at[c, pl.ds(s * RPT, RPT)])

    return pl.kernel(body, mesh=mesh, out_type=out_type, scratch_types=scratch,
                     compiler_params=pltpu.CompilerParams(
                         use_tc_tiling_on_sc=False))


def _make_sc_counts():
    mesh = plsc.VectorSubcoreMesh(core_axis_name="c", subcore_axis_name="s")
    out_type = jax.ShapeDtypeStruct((2, NP, 16), jnp.float32)
    scratch = [
        pltpu.VMEM((K, CH), jnp.int32),
        pltpu.VMEM((CH, 16), jnp.float32),
        pltpu.VMEM((ZR, 16), jnp.float32),
        pltpu.VMEM_SHARED((NP, 16), jnp.float32),
    ]

    def body(dst_hbm, cnt_hbm, dsts_v, ones_v, zc_v, cnt_sh):
        c = lax.axis_index("c")
        s = lax.axis_index("s")
        w = c * 16 + s

        def ofill(i, carry):
            ones_v[i, :] = jnp.full((16,), 1.0, jnp.float32)
            return carry
        lax.fori_loop(0, CH, ofill, 0)

        def zcfill(i, carry):
            zc_v[i, :] = jnp.zeros((16,), jnp.float32)
            return carry
        lax.fori_loop(0, ZR, zcfill, 0)
        for t in range(RPT // ZR):
            pltpu.sync_copy(zc_v, cnt_sh.at[pl.ds(s * RPT + t * ZR, ZR)])
        plsc.subcore_barrier()

        pltpu.sync_copy(dst_hbm.at[w], dsts_v)

        def chunk(k, carry):
            pltpu.sync_copy(ones_v, cnt_sh.at[dsts_v.at[k]], add=True)
            return carry
        lax.fori_loop(0, K, chunk, 0)
        plsc.subcore_barrier()

        pltpu.sync_copy(cnt_sh.at[pl.ds(s * RPT, RPT)],
                        cnt_hbm.at[c, pl.ds(s * RPT, RPT)])

    return pl.kernel(body, mesh=mesh, out_type=out_type, scratch_types=scratch,
                     compiler_params=pltpu.CompilerParams(
                         use_tc_tiling_on_sc=False))


_sc_counts = _make_sc_counts()
_sc64 = _make_sc_agg64()
_sc128d = _make_sc_agg64(D, dual=True)


def _dotT(a, w):
    return lax.dot_general(a, w, (((1,), (1,)), ((), ())),
                           precision=lax.Precision.HIGHEST,
                           preferred_element_type=jnp.float32)


def _mean1(p_ref, inv):
    return (p_ref[0] + p_ref[1]) * inv


R = 1024
G = NP // R

_vspec = pl.BlockSpec((2, R, H), lambda i: (0, i, 0))
_ispec = pl.BlockSpec((R,), lambda i: (i,))


def _make_tc_mm(din, dout):
    def mm_body(x_ref, w_ref, o_ref):
        o_ref[...] = _dotT(x_ref[...], w_ref[...])

    return pl.pallas_call(
        mm_body,
        grid=(G,),
        in_specs=[pl.BlockSpec((R, din), lambda i: (i, 0)),
                  pl.BlockSpec((dout, din), lambda i: (0, 0))],
        out_specs=[pl.BlockSpec((R, dout), lambda i: (i, 0))],
        out_shape=[jax.ShapeDtypeStruct((NP, dout), jnp.float32)],
    )


_mm_d_d = _make_tc_mm(D, D)
_mm_d_h = _make_tc_mm(D, H)
_mm_h_d = _make_tc_mm(H, D)


def _tc0_body(pa_ref, pb_ref, cp_ref, xw_ref, wl0_ref, bl0_ref,
              wl1_ref, y0_ref, t1_ref, inv_ref):
    cnt = cp_ref[0, :, 0] + cp_ref[1, :, 0]
    inv = 1.0 / jnp.maximum(cnt, 1.0)
    ma = _mean1(pa_ref, inv[:, None])
    mb = _mean1(pb_ref, inv[:, None])
    y = (_dotT(ma, wl0_ref[:, :H]) + _dotT(mb, wl0_ref[:, H:])
         + bl0_ref[...][None, :] + xw_ref[...])
    y = jnp.maximum(y, 0.0)
    y0_ref[...] = y
    t1_ref[...] = _dotT(y, wl1_ref[...])
    inv_ref[...] = inv


_tc0 = pl.pallas_call(
    _tc0_body,
    grid=(G,),
    in_specs=[
        _vspec,
        _vspec,
        pl.BlockSpec((2, R, 16), lambda i: (0, i, 0)),
        pl.BlockSpec((R, D), lambda i: (i, 0)),
        pl.BlockSpec((D, D), lambda i: (0, 0)),
        pl.BlockSpec((D,), lambda i: (0,)),
        pl.BlockSpec((H, D), lambda i: (0, 0)),
    ],
    out_specs=[pl.BlockSpec((R, D), lambda i: (i, 0)),
               pl.BlockSpec((R, H), lambda i: (i, 0)),
               _ispec],
    out_shape=[jax.ShapeDtypeStruct((NP, D), jnp.float32),
               jax.ShapeDtypeStruct((NP, H), jnp.float32),
               jax.ShapeDtypeStruct((NP,), jnp.float32)],
)


def _tc1_body(p_ref, inv_ref, yw_ref, bl1_ref, h1_ref):
    m = _mean1(p_ref, inv_ref[...][:, None])
    h1_ref[...] = m + bl1_ref[...][None, :] + yw_ref[...]


_tc1 = pl.pallas_call(
    _tc1_body,
    grid=(G,),
    in_specs=[
        _vspec,
        _ispec,
        pl.BlockSpec((R, H), lambda i: (i, 0)),
        pl.BlockSpec((H,), lambda i: (0,)),
    ],
    out_specs=[pl.BlockSpec((R, H), lambda i: (i, 0))],
    out_shape=[jax.ShapeDtypeStruct((NP, H), jnp.float32)],
)


def _tc2_body(p_ref, inv_ref, hw_ref, wl2_ref, bl2_ref, h2_ref):
    m = _mean1(p_ref, inv_ref[...][:, None])
    y = _dotT(m, wl2_ref[...]) + bl2_ref[...][None, :] + hw_ref[...]
    h2_ref[...] = jnp.maximum(y, 0.0)


_tc2 = pl.pallas_call(
    _tc2_body,
    grid=(G,),
    in_specs=[
        _vspec,
        _ispec,
        pl.BlockSpec((R, D), lambda i: (i, 0)),
        pl.BlockSpec((D, H), lambda i: (0, 0)),
        pl.BlockSpec((D,), lambda i: (0,)),
    ],
    out_specs=[pl.BlockSpec((R, D), lambda i: (i, 0))],
    out_shape=[jax.ShapeDtypeStruct((NP, D), jnp.float32)],
)


def _tc3_body(pa_ref, pb_ref, inv_ref, h2w_ref, wl3_ref, bl3_ref, o_ref):
    inv = inv_ref[...][:, None]
    ma = _mean1(pa_ref, inv)
    mb = _mean1(pb_ref, inv)
    o_ref[...] = (_dotT(ma, wl3_ref[:, :H]) + _dotT(mb, wl3_ref[:, H:])
                  + bl3_ref[...][None, :] + h2w_ref[...])


_tc3 = pl.pallas_call(
    _tc3_body,
    grid=(G,),
    in_specs=[
        _vspec,
        _vspec,
        _ispec,
        pl.BlockSpec((R, D), lambda i: (i, 0)),
        pl.BlockSpec((D, D), lambda i: (0, 0)),
        pl.BlockSpec((D,), lambda i: (0,)),
    ],
    out_specs=[pl.BlockSpec((R, D), lambda i: (i, 0))],
    out_shape=[jax.ShapeDtypeStruct((NP, D), jnp.float32)],
)


def _first(res):
    return res[0] if isinstance(res, (list, tuple)) else res


def kernel(x, edge_index, Wl0, bl0, Wr0, Wl1, bl1, Wr1, Wl2, bl2, Wr2,
           Wl3, bl3, Wr3):
    pad = EP - E
    src3 = jnp.concatenate(
        [edge_index[0], jnp.zeros((pad,), jnp.int32)]).reshape(NW, K, CH)
    dst3 = jnp.concatenate(
        [edge_index[1], jnp.full((pad,), N, jnp.int32)]).reshape(NW, K, CH)
    x_p = jnp.pad(x, ((0, NP - N), (0, 0)))

    cp = _first(_sc_counts(dst3))
    xw = _first(_mm_d_d(x_p, Wr0))
    p0a, p0b = _sc128d(x_p, src3, dst3)
    y0, t1, inv = _tc0(p0a, p0b, cp, xw, Wl0, bl0, Wl1)
    yw = _first(_mm_d_h(y0, Wr1))
    p1 = _first(_sc64(t1, src3, dst3))
    h1 = _first(_tc1(p1, inv, yw, bl1))
    hw = _first(_mm_h_d(h1, Wr2))
    p2 = _first(_sc64(h1, src3, dst3))
    h2 = _first(_tc2(p2, inv, hw, Wl2, bl2))
    h2w = _first(_mm_d_d(h2, Wr3))
    p3a, p3b = _sc128d(h2, src3, dst3)
    return _first(_tc3(p3a, p3b, inv, h2w, Wl3, bl3))[:N]


# speedup vs baseline: 11.0636x; 1.0014x over previous
import functools

import jax
import jax.numpy as jnp
from jax import lax
from jax.experimental import pallas as pl
from jax.experimental.pallas import tpu as pltpu
from jax.experimental.pallas import tpu_sc as plsc

N = 10000
NP = 10240
D = 128
H = 64
E = 320000
CH = 128
NW = 32
K = 80
EP = NW * K * CH
GC = 4
NG = K // GC
RPT = NP // 16
ZR = 32


def _make_sc_agg64(srcw: int = H, dual: bool = False):
    F = H
    mesh = plsc.VectorSubcoreMesh(core_axis_name="c", subcore_axis_name="s")
    if dual:
        out_type = [jax.ShapeDtypeStruct((2, NP, F), jnp.float32),
                    jax.ShapeDtypeStruct((2, NP, F), jnp.float32)]
    else:
        out_type = jax.ShapeDtypeStruct((2, NP, F), jnp.float32)
    scratch = [
        pltpu.VMEM((GC, CH), jnp.int32),
        pltpu.VMEM((GC, CH), jnp.int32),
        pltpu.VMEM((GC, CH), jnp.int32),
        pltpu.VMEM((GC, CH), jnp.int32),
        pltpu.VMEM((CH, F), jnp.float32),
        pltpu.VMEM((CH, F), jnp.float32),
        pltpu.VMEM((CH, F), jnp.float32),
        pltpu.VMEM((CH, F), jnp.float32),
        pltpu.VMEM((ZR, F), jnp.float32),
        pltpu.VMEM_SHARED((NP, F), jnp.float32),
        pltpu.VMEM_SHARED((NP, F), jnp.float32),
        pltpu.SemaphoreType.DMA,
        pltpu.SemaphoreType.DMA,
        pltpu.SemaphoreType.DMA,
        pltpu.SemaphoreType.DMA,
        pltpu.SemaphoreType.DMA,
        pltpu.SemaphoreType.DMA,
        pltpu.SemaphoreType.DMA,
        pltpu.SemaphoreType.DMA,
        pltpu.SemaphoreType.DMA,
    ]
    def body(h_hbm, src_hbm, dst_hbm, *rest):
        if dual:
            (outA_hbm, outB_hbm, srcg0, dstg0, srcg1, dstg1,
             rows0, rows1, rows2, rows3, zb_v, feat_sh, acc_sh,
             gsem0, gsem1, gsem2, gsem3, ssem0, ssem1, ssem2, ssem3,
             zsem) = rest
            passes = ((0, outA_hbm), (H, outB_hbm))
        else:
            (out_hbm, srcg0, dstg0, srcg1, dstg1,
             rows0, rows1, rows2, rows3, zb_v, feat_sh, acc_sh,
             gsem0, gsem1, gsem2, gsem3, ssem0, ssem1, ssem2, ssem3,
             zsem) = rest
            passes = ((0, out_hbm),)
        c = lax.axis_index("c")
        s = lax.axis_index("s")
        w = c * 16 + s

        idxbufs = ((srcg0, dstg0), (srcg1, dstg1))
        rowsb = (rows0, rows1, rows2, rows3)
        gsems = (gsem0, gsem1, gsem2, gsem3)
        ssems = (ssem0, ssem1, ssem2, ssem3)

        def zfill(i, carry):
            for j in range(F // 16):
                zb_v[i, pl.ds(j * 16, 16)] = jnp.zeros((16,), jnp.float32)
            return carry
        lax.fori_loop(0, ZR, zfill, 0)

        for col0, out_hbm in passes:
            if srcw == F:
                src_slab = h_hbm.at[pl.ds(s * RPT, RPT)]
            else:
                src_slab = h_hbm.at[pl.ds(s * RPT, RPT), pl.ds(col0, F)]
            pltpu.async_copy(src_slab, feat_sh.at[pl.ds(s * RPT, RPT)], zsem)
            for t in range(RPT // ZR):
                pltpu.async_copy(zb_v, acc_sh.at[pl.ds(s * RPT + t * ZR, ZR)],
                                 zsem)
            pltpu.make_async_copy(src_slab,
                                  feat_sh.at[pl.ds(s * RPT, RPT)], zsem).wait()
            for t in range(RPT // ZR):
                pltpu.make_async_copy(
                    zb_v, acc_sh.at[pl.ds(s * RPT, ZR)], zsem).wait()
            plsc.subcore_barrier()

            pltpu.sync_copy(src_hbm.at[w, pl.ds(0, GC)], srcg0)
            pltpu.sync_copy(dst_hbm.at[w, pl.ds(0, GC)], dstg0)
            pltpu.async_copy(feat_sh.at[srcg0.at[0]], rows0, gsem0)
            pltpu.async_copy(feat_sh.at[srcg0.at[1]], rows1, gsem1)

            def iter_body(i, carry):
                for gp in range(2):
                    srcg, dstg = idxbufs[gp]
                    nsrcg, ndstg = idxbufs[1 - gp]
                    g = 2 * i + gp
                    for j in range(GC):
                        k = g * GC + j
                        rows_v = rowsb[j]
                        nx = (j + 2) % 4
                        pltpu.make_async_copy(
                            feat_sh.at[srcg.at[j]], rows_v, gsems[j]).wait()
                        pltpu.async_copy(
                            rows_v, acc_sh.at[dstg.at[j]], ssems[j], add=True)

                        @pl.when(k >= 2)
                        def _():
                            if j < 2:
                                pltpu.make_async_copy(
                                    rowsb[nx], acc_sh.at[ndstg.at[nx]],
                                    ssems[nx]).wait()
                            else:
                                pltpu.make_async_copy(
                                    rowsb[nx], acc_sh.at[dstg.at[nx]],
                                    ssems[nx]).wait()

                        if j == 2:
                            @pl.when(g + 1 < NG)
                            def _():
                                pltpu.sync_copy(
                                    src_hbm.at[w, pl.ds((g + 1) * GC, GC)],
                                    nsrcg)
                                pltpu.sync_copy(
                                    dst_hbm.at[w, pl.ds((g + 1) * GC, GC)],
                                    ndstg)

                        @pl.when(k + 2 < K)
                        def _():
                            if j < GC - 2:
                                pltpu.async_copy(
                                    feat_sh.at[srcg.at[j + 2]], rowsb[nx],
                                    gsems[nx])
                            else:
                                pltpu.async_copy(
                                    feat_sh.at[nsrcg.at[j + 2 - GC]],
                                    rowsb[nx], gsems[nx])
                return carry
            lax.fori_loop(0, NG // 2, iter_body, 0)
            pltpu.make_async_copy(rows2, acc_sh.at[dstg1.at[2]], ssem2).wait()
            pltpu.make_async_copy(rows3, acc_sh.at[dstg1.at[3]], ssem3).wait()
            plsc.subcore_barrier()

            pltpu.sync_copy(acc_sh.at[pl.ds(s * RPT, RPT)],
                            out_hbm.at[c, pl.ds(s * RPT, RPT)])

    return pl.kernel(body, mesh=mesh, out_type=out_type, scratch_types=scratch,
                     compiler_params=pltpu.CompilerParams(
                         use_tc_tiling_on_sc=False))


def _make_sc_counts():
    mesh = plsc.VectorSubcoreMesh(core_axis_name="c", subcore_axis_name="s")
    out_type = jax.ShapeDtypeStruct((2, NP, 16), jnp.float32)
    scratch = [
        pltpu.VMEM((K, CH), jnp.int32),
        pltpu.VMEM((CH, 16), jnp.float32),
        pltpu.VMEM((ZR, 16), jnp.float32),
        pltpu.VMEM_SHARED((NP, 16), jnp.float32),
    ]

    def body(dst_hbm, cnt_hbm, dsts_v, ones_v, zc_v, cnt_sh):
        c = lax.axis_index("c")
        s = lax.axis_index("s")
        w = c * 16 + s

        def ofill(i, carry):
            ones_v[i, :] = jnp.full((16,), 1.0, jnp.float32)
            return carry
        lax.fori_loop(0, CH, ofill, 0)

        def zcfill(i, carry):
            zc_v[i, :] = jnp.zeros((16,), jnp.float32)
            return carry
        lax.fori_loop(0, ZR, zcfill, 0)
        for t in range(RPT // ZR):
            pltpu.sync_copy(zc_v, cnt_sh.at[pl.ds(s * RPT + t * ZR, ZR)])
        plsc.subcore_barrier()

        pltpu.sync_copy(dst_hbm.at[w], dsts_v)

        def chunk(k, carry):
            pltpu.sync_copy(ones_v, cnt_sh.at[dsts_v.at[k]], add=True)
            return carry
        lax.fori_loop(0, K, chunk, 0)
        plsc.subcore_barrier()

        pltpu.sync_copy(cnt_sh.at[pl.ds(s * RPT, RPT)],
                        cnt_hbm.at[c, pl.ds(s * RPT, RPT)])

    return pl.kernel(body, mesh=mesh, out_type=out_type, scratch_types=scratch,
                     compiler_params=pltpu.CompilerParams(
                         use_tc_tiling_on_sc=False))


_sc_counts = _make_sc_counts()
_sc64 = _make_sc_agg64()
_sc128d = _make_sc_agg64(D, dual=True)


def _dotT(a, w):
    return lax.dot_general(a, w, (((1,), (1,)), ((), ())),
                           precision=lax.Precision.HIGHEST,
                           preferred_element_type=jnp.float32)


def _mean1(p_ref, inv):
    return (p_ref[0] + p_ref[1]) * inv


R = 1024
G = NP // R

_vspec = pl.BlockSpec((2, R, H), lambda i: (0, i, 0))
_ispec = pl.BlockSpec((R,), lambda i: (i,))


def _make_tc_mm(din, dout):
    def mm_body(x_ref, w_ref, o_ref):
        o_ref[...] = _dotT(x_ref[...], w_ref[...])

    return pl.pallas_call(
        mm_body,
        grid=(G,),
        in_specs=[pl.BlockSpec((R, din), lambda i: (i, 0)),
                  pl.BlockSpec((dout, din), lambda i: (0, 0))],
        out_specs=[pl.BlockSpec((R, dout), lambda i: (i, 0))],
        out_shape=[jax.ShapeDtypeStruct((NP, dout), jnp.float32)],
    )


_mm_d_d = _make_tc_mm(D, D)
_mm_d_h = _make_tc_mm(D, H)
_mm_h_d = _make_tc_mm(H, D)


def _tc0_body(pa_ref, pb_ref, cp_ref, xw_ref, wl0_ref, bl0_ref,
              wl1_ref, y0_ref, t1_ref, inv_ref):
    cnt = cp_ref[0, :, 0] + cp_ref[1, :, 0]
    inv = 1.0 / jnp.maximum(cnt, 1.0)
    ma = _mean1(pa_ref, inv[:, None])
    mb = _mean1(pb_ref, inv[:, None])
    y = (_dotT(ma, wl0_ref[:, :H]) + _dotT(mb, wl0_ref[:, H:])
         + bl0_ref[...][None, :] + xw_ref[...])
    y = jnp.maximum(y, 0.0)
    y0_ref[...] = y
    t1_ref[...] = _dotT(y, wl1_ref[...])
    inv_ref[...] = inv


_tc0 = pl.pallas_call(
    _tc0_body,
    grid=(G,),
    in_specs=[
        _vspec,
        _vspec,
        pl.BlockSpec((2, R, 16), lambda i: (0, i, 0)),
        pl.BlockSpec((R, D), lambda i: (i, 0)),
        pl.BlockSpec((D, D), lambda i: (0, 0)),
        pl.BlockSpec((D,), lambda i: (0,)),
        pl.BlockSpec((H, D), lambda i: (0, 0)),
    ],
    out_specs=[pl.BlockSpec((R, D), lambda i: (i, 0)),
               pl.BlockSpec((R, H), lambda i: (i, 0)),
               _ispec],
    out_shape=[jax.ShapeDtypeStruct((NP, D), jnp.float32),
               jax.ShapeDtypeStruct((NP, H), jnp.float32),
               jax.ShapeDtypeStruct((NP,), jnp.float32)],
)


def _tc1_body(p_ref, inv_ref, yw_ref, bl1_ref, h1_ref):
    m = _mean1(p_ref, inv_ref[...][:, None])
    h1_ref[...] = m + bl1_ref[...][None, :] + yw_ref[...]


_tc1 = pl.pallas_call(
    _tc1_body,
    grid=(G,),
    in_specs=[
        _vspec,
        _ispec,
        pl.BlockSpec((R, H), lambda i: (i, 0)),
        pl.BlockSpec((H,), lambda i: (0,)),
    ],
    out_specs=[pl.BlockSpec((R, H), lambda i: (i, 0))],
    out_shape=[jax.ShapeDtypeStruct((NP, H), jnp.float32)],
)


def _tc2_body(p_ref, inv_ref, hw_ref, wl2_ref, bl2_ref, h2_ref):
    m = _mean1(p_ref, inv_ref[...][:, None])
    y = _dotT(m, wl2_ref[...]) + bl2_ref[...][None, :] + hw_ref[...]
    h2_ref[...] = jnp.maximum(y, 0.0)


_tc2 = pl.pallas_call(
    _tc2_body,
    grid=(G,),
    in_specs=[
        _vspec,
        _ispec,
        pl.BlockSpec((R, D), lambda i: (i, 0)),
        pl.BlockSpec((D, H), lambda i: (0, 0)),
        pl.BlockSpec((D,), lambda i: (0,)),
    ],
    out_specs=[pl.BlockSpec((R, D), lambda i: (i, 0))],
    out_shape=[jax.ShapeDtypeStruct((NP, D), jnp.float32)],
)


def _tc3_body(pa_ref, pb_ref, inv_ref, h2w_ref, wl3_ref, bl3_ref, o_ref):
    inv = inv_ref[...][:, None]
    ma = _mean1(pa_ref, inv)
    mb = _mean1(pb_ref, inv)
    o_ref[...] = (_dotT(ma, wl3_ref[:, :H]) + _dotT(mb, wl3_ref[:, H:])
                  + bl3_ref[...][None, :] + h2w_ref[...])


_tc3 = pl.pallas_call(
    _tc3_body,
    grid=(G,),
    in_specs=[
        _vspec,
        _vspec,
        _ispec,
        pl.BlockSpec((R, D), lambda i: (i, 0)),
        pl.BlockSpec((D, D), lambda i: (0, 0)),
        pl.BlockSpec((D,), lambda i: (0,)),
    ],
    out_specs=[pl.BlockSpec((R, D), lambda i: (i, 0))],
    out_shape=[jax.ShapeDtypeStruct((NP, D), jnp.float32)],
)


def _first(res):
    return res[0] if isinstance(res, (list, tuple)) else res


def kernel(x, edge_index, Wl0, bl0, Wr0, Wl1, bl1, Wr1, Wl2, bl2, Wr2,
           Wl3, bl3, Wr3):
    pad = EP - E
    src3 = jnp.concatenate(
        [edge_index[0], jnp.zeros((pad,), jnp.int32)]).reshape(NW, K, CH)
    dst3 = jnp.concatenate(
        [edge_index[1], jnp.full((pad,), N, jnp.int32)]).reshape(NW, K, CH)
    x_p = jnp.pad(x, ((0, NP - N), (0, 0)))

    cp = _first(_sc_counts(dst3))
    xw = _first(_mm_d_d(x_p, Wr0))
    p0a, p0b = _sc128d(x_p, src3, dst3)
    y0, t1, inv = _tc0(p0a, p0b, cp, xw, Wl0, bl0, Wl1)
    yw = _first(_mm_d_h(y0, Wr1))
    p1 = _first(_sc64(t1, src3, dst3))
    h1 = _first(_tc1(p1, inv, yw, bl1))
    hw = _first(_mm_h_d(h1, Wr2))
    p2 = _first(_sc64(h1, src3, dst3))
    h2 = _first(_tc2(p2, inv, hw, Wl2, bl2))
    h2w = _first(_mm_d_d(h2, Wr3))
    p3a, p3b = _sc128d(h2, src3, dst3)
    return _first(_tc3(p3a, p3b, inv, h2w, Wl3, bl3))[:N]
